# baseline (speedup 1.0000x reference)
import functools

import jax
import jax.numpy as jnp
from jax import lax
from jax.experimental import pallas as pl
from jax.experimental.pallas import tpu as pltpu
from jax.experimental.pallas import tpu_sc as plsc

N, E, D, H, P, ED = 10000, 320000, 128, 128, 3, 16

NC = 2
NS = 16
NW = NC * NS
N_PAD = 10240
ROWS_PER_TILE = N_PAD // NS
E_PAD = 10240 * NW
EW = E_PAD // NW
CG = 160
NCG = EW // CG
CS = 128
NCS = EW // CS
IDX_ROWS = EW // CS
REL_CHUNK = 512
NREL = EW // REL_CHUNK

_f32 = jnp.float32

BE = 4096
BN = 2000


def _sigmoid(v):
    return 1.0 / (1.0 + jnp.exp(-v))


def _silu(v):
    return v * _sigmoid(v)


_SC_MESH = plsc.VectorSubcoreMesh(core_axis_name="c", subcore_axis_name="s",
                                  num_cores=NC, num_subcores=NS)


@functools.partial(
    pl.kernel,
    out_type=jax.ShapeDtypeStruct((E_PAD, 16), _f32),
    mesh=_SC_MESH,
    compiler_params=pltpu.CompilerParams(needs_layout_passes=False),
    scratch_types=[
        pltpu.VMEM((P * N,), _f32),
        pltpu.VMEM((EW,), jnp.int32),
        pltpu.VMEM((EW,), jnp.int32),
        pltpu.VMEM((REL_CHUNK, 16), _f32),
    ],
)
def _sc_rel(posf_hbm, dst_hbm, src_hbm, rel_hbm, posv, di, si, rbuf):
    wid = lax.axis_index("c") * NS + lax.axis_index("s")
    base0 = wid * EW
    pltpu.sync_copy(posf_hbm, posv)
    pltpu.sync_copy(dst_hbm.at[pl.ds(base0, EW)], di)
    pltpu.sync_copy(src_hbm.at[pl.ds(base0, EW)], si)
    zero16 = jnp.zeros((16,), _f32)

    @plsc.parallel_loop(0, REL_CHUNK)
    def _(r):
        rbuf[r, :] = zero16

    lanes = lax.iota(jnp.int32, 16)

    def chunk(j, carry):
        @plsc.parallel_loop(0, REL_CHUNK // 16, unroll=2)
        def _(v):
            e0 = j * REL_CHUNK + v * 16
            dstv = di[pl.ds(e0, 16)]
            srcv = si[pl.ds(e0, 16)]
            rows = v * 16 + lanes
            for comp in range(P):
                pdc = plsc.load_gather(posv, [dstv + comp * N])
                psc = plsc.load_gather(posv, [srcv + comp * N])
                cols = jnp.full((16,), comp, jnp.int32)
                plsc.store_scatter(rbuf, [rows, cols], pdc - psc)

        pltpu.sync_copy(rbuf, rel_hbm.at[pl.ds(base0 + j * REL_CHUNK, REL_CHUNK)])
        return carry

    lax.fori_loop(0, NREL, chunk, 0)


@functools.partial(
    pl.kernel,
    out_type=jax.ShapeDtypeStruct((E_PAD, H), _f32),
    mesh=_SC_MESH,
    compiler_params=pltpu.CompilerParams(needs_layout_passes=False),
    scratch_types=[
        pltpu.VMEM((EW,), jnp.int32),
        pltpu.VMEM((EW,), jnp.int32),
        pltpu.VMEM((2, CG, H), _f32),
        pltpu.VMEM((2, CG, H), _f32),
        pltpu.SemaphoreType.DMA,
        pltpu.SemaphoreType.DMA,
    ],
)
def _sc_gather(xd_hbm, xs_hbm, dst_hbm, src_hbm,
               pre0_hbm, di, si, bufd, bufs, sem0, sem1):
    wid = lax.axis_index("c") * NS + lax.axis_index("s")
    base0 = wid * EW
    pltpu.sync_copy(dst_hbm.at[pl.ds(base0, EW)], di)
    pltpu.sync_copy(src_hbm.at[pl.ds(base0, EW)], si)
    sems = (sem0, sem1)

    def gathers(j, slot, sem):
        e0 = j * CG
        cpd = pltpu.make_async_copy(
            xd_hbm.at[di.at[pl.ds(e0, CG)]], bufd.at[slot], sem)
        cps = pltpu.make_async_copy(
            xs_hbm.at[si.at[pl.ds(e0, CG)]], bufs.at[slot], sem)
        return cpd, cps

    for j0 in range(2):
        cpd, cps = gathers(j0, j0, sems[j0])
        cpd.start()
        cps.start()

    def pair(p, carry):
        for s2 in range(2):
            j = p * 2 + s2
            cpd, cps = gathers(j, s2, sems[s2])
            cpd.wait()
            cps.wait()

            @plsc.parallel_loop(0, CG, unroll=4)
            def _(r):
                for k in range(H // 16):
                    sl = pl.ds(k * 16, 16)
                    bufd[s2, r, sl] = bufd[s2, r, sl] + bufs[s2, r, sl]

            pltpu.sync_copy(bufd.at[s2],
                            pre0_hbm.at[pl.ds(base0 + j * CG, CG)])
            jj = j + 2

            @pl.when(jj < NCG)
            def _():
                cpd2, cps2 = gathers(jj, s2, sems[s2])
                cpd2.start()
                cps2.start()
        return carry

    lax.fori_loop(0, NCG // 2, pair, 0)


@functools.partial(
    pl.kernel,
    out_type=jax.ShapeDtypeStruct((NC, N_PAD, H), _f32),
    mesh=_SC_MESH,
    compiler_params=pltpu.CompilerParams(needs_layout_passes=False),
    scratch_types=[
        pltpu.VMEM_SHARED((N_PAD, H), _f32),
        pltpu.VMEM((IDX_ROWS, CS), jnp.int32),
        pltpu.VMEM((2, CS, H), _f32),
        pltpu.SemaphoreType.DMA,
        pltpu.SemaphoreType.DMA,
    ],
)
def _sc_scatter(m_hbm, dst2d_hbm, z_hbm, agg_hbm, shared, di2, mbuf,
                sem0, sem1):
    c = lax.axis_index("c")
    s = lax.axis_index("s")
    wid = c * NS + s
    rows = pl.ds(s * ROWS_PER_TILE, ROWS_PER_TILE)
    pltpu.sync_copy(z_hbm.at[rows], shared.at[rows])
    pltpu.sync_copy(dst2d_hbm.at[pl.ds(wid * IDX_ROWS, IDX_ROWS)], di2)
    plsc.subcore_barrier()
    sems = (sem0, sem1)
    base0 = wid * EW

    def load(j, slot, sem):
        return pltpu.make_async_copy(
            m_hbm.at[pl.ds(base0 + j * CS, CS)], mbuf.at[slot], sem)

    for j0 in range(2):
        load(j0, j0, sems[j0]).start()

    def pair(p, carry):
        for s2 in range(2):
            j = p * 2 + s2
            load(j, s2, sems[s2]).wait()
            pltpu.sync_copy(mbuf.at[s2], shared.at[di2.at[j]], add=True)
            jj = j + 2

            @pl.when(jj < NCS)
            def _():
                load(jj, s2, sems[s2]).start()
        return carry

    lax.fori_loop(0, NCS // 2, pair, 0)
    plsc.subcore_barrier()
    pltpu.sync_copy(shared.at[rows], agg_hbm.at[c, rows])


def _w_spec(shape):
    return pl.BlockSpec(shape, lambda i: (0,) * len(shape))


def _proj_body(x_ref, wd_ref, ws_ref, xd_ref, xs_ref):
    v = x_ref[...]
    xd_ref[...] = jnp.dot(v, wd_ref[...], preferred_element_type=_f32)
    xs_ref[...] = jnp.dot(v, ws_ref[...], preferred_element_type=_f32)


def _tc_proj(x, wd, ws):
    return pl.pallas_call(
        _proj_body,
        grid=(N // BN,),
        in_specs=[
            pl.BlockSpec((BN, D), lambda i: (i, 0)),
            _w_spec((D, H)),
            _w_spec((D, H)),
        ],
        out_specs=[
            pl.BlockSpec((BN, H), lambda i: (i, 0)),
            pl.BlockSpec((BN, H), lambda i: (i, 0)),
        ],
        out_shape=[
            jax.ShapeDtypeStruct((N, H), _f32),
            jax.ShapeDtypeStruct((N, H), _f32),
        ],
    )(x, wd, ws)


def _edge1_body(pre0_ref, rel_ref, ea_ref, wdist_ref, we_ref, b0_ref,
                w1_ref, b1_ref, lmask_ref, m_ref):
    rel = rel_ref[...]
    dist2 = jnp.sum(rel * rel * lmask_ref[...], axis=-1, keepdims=True)
    pre = (pre0_ref[...] + dist2 * wdist_ref[...] + b0_ref[...]
           + jnp.dot(ea_ref[...], we_ref[...], preferred_element_type=_f32))
    m1 = _silu(pre)
    z = jnp.dot(m1, w1_ref[...], preferred_element_type=_f32) + b1_ref[...]
    m_ref[...] = _silu(z)


def _tc_edge1(pre0, rel, ea, wdist, we, b0, w1, b1, lmask):
    return pl.pallas_call(
        _edge1_body,
        grid=(E_PAD // BE,),
        in_specs=[
            pl.BlockSpec((BE, H), lambda i: (i, 0)),
            pl.BlockSpec((BE, 16), lambda i: (i, 0)),
            pl.BlockSpec((BE, ED), lambda i: (i, 0)),
            _w_spec((1, H)),
            _w_spec((ED, H)),
            _w_spec((1, H)),
            _w_spec((H, H)),
            _w_spec((1, H)),
            _w_spec((1, 16)),
        ],
        out_specs=pl.BlockSpec((BE, H), lambda i: (i, 0)),
        out_shape=jax.ShapeDtypeStruct((E_PAD, H), _f32),
    )(pre0, rel, ea, wdist, we, b0, w1, b1, lmask)


def _edge2_body(pre0_ref, rel_ref, ea_ref, wdist_ref, we_ref, b0_ref,
                w1_ref, b1_ref, wp0_ref, bp0_ref, wp1_ref, bp1_ref,
                lmask_ref, oh3_ref, m_ref, relw_ref):
    rel = rel_ref[...]
    dist2 = jnp.sum(rel * rel * lmask_ref[...], axis=-1, keepdims=True)
    pre = (pre0_ref[...] + dist2 * wdist_ref[...] + b0_ref[...]
           + jnp.dot(ea_ref[...], we_ref[...], preferred_element_type=_f32))
    m1 = _silu(pre)
    z = jnp.dot(m1, w1_ref[...], preferred_element_type=_f32) + b1_ref[...]
    m = _silu(z)
    m_ref[...] = m
    t = jnp.dot(m, wp0_ref[...], preferred_element_type=_f32) + bp0_ref[...]
    t = _silu(t)
    w2 = jnp.sum(t * wp1_ref[...], axis=-1, keepdims=True) + bp1_ref[:, :1]
    relw = jnp.concatenate([rel * w2, jnp.zeros((BE, H - 16), _f32)], axis=1)
    relw_ref[...] = relw + oh3_ref[...]


def _tc_edge2(pre0, rel, ea, wdist, we, b0, w1, b1, wp0, bp0, wp1, bp1,
              lmask, oh3):
    return pl.pallas_call(
        _edge2_body,
        grid=(E_PAD // BE,),
        in_specs=[
            pl.BlockSpec((BE, H), lambda i: (i, 0)),
            pl.BlockSpec((BE, 16), lambda i: (i, 0)),
            pl.BlockSpec((BE, ED), lambda i: (i, 0)),
            _w_spec((1, H)),
            _w_spec((ED, H)),
            _w_spec((1, H)),
            _w_spec((H, H)),
            _w_spec((1, H)),
            _w_spec((H, H)),
            _w_spec((1, H)),
            _w_spec((1, H)),
            _w_spec((1, H)),
            _w_spec((1, 16)),
            _w_spec((1, H)),
        ],
        out_specs=[
            pl.BlockSpec((BE, H), lambda i: (i, 0)),
            pl.BlockSpec((BE, H), lambda i: (i, 0)),
        ],
        out_shape=[
            jax.ShapeDtypeStruct((E_PAD, H), _f32),
            jax.ShapeDtypeStruct((E_PAD, H), _f32),
        ],
    )(pre0, rel, ea, wdist, we, b0, w1, b1, wp0, bp0, wp1, bp1, lmask, oh3)


def _node1_body(x_ref, aggp_ref, wn0x_ref, wn0a_ref, bn0_ref, wn1_ref,
                bn1_ref, wd2_ref, ws2_ref, h_ref, xd2_ref, xs2_ref):
    agg = aggp_ref[0] + aggp_ref[1]
    t = (jnp.dot(x_ref[...], wn0x_ref[...], preferred_element_type=_f32)
         + jnp.dot(agg, wn0a_ref[...], preferred_element_type=_f32)
         + bn0_ref[...])
    t = _silu(t)
    hv = jnp.dot(t, wn1_ref[...], preferred_element_type=_f32) + bn1_ref[...]
    h_ref[...] = hv
    xd2_ref[...] = jnp.dot(hv, wd2_ref[...], preferred_element_type=_f32)
    xs2_ref[...] = jnp.dot(hv, ws2_ref[...], preferred_element_type=_f32)


def _tc_node1(x, aggp, wn0x, wn0a, bn0, wn1, bn1, wd2, ws2):
    return pl.pallas_call(
        _node1_body,
        grid=(N // BN,),
        in_specs=[
            pl.BlockSpec((BN, D), lambda i: (i, 0)),
            pl.BlockSpec((NC, BN, H), lambda i: (0, i, 0)),
            _w_spec((D, H)),
            _w_spec((H, H)),
            _w_spec((1, H)),
            _w_spec((H, H)),
            _w_spec((1, H)),
            _w_spec((H, H)),
            _w_spec((H, H)),
        ],
        out_specs=[
            pl.BlockSpec((BN, H), lambda i: (i, 0)),
            pl.BlockSpec((BN, H), lambda i: (i, 0)),
            pl.BlockSpec((BN, H), lambda i: (i, 0)),
        ],
        out_shape=[
            jax.ShapeDtypeStruct((N, H), _f32),
            jax.ShapeDtypeStruct((N, H), _f32),
            jax.ShapeDtypeStruct((N, H), _f32),
        ],
    )(x, aggp, wn0x, wn0a, bn0, wn1, bn1, wd2, ws2)


def _node2_body(h_ref, aggp_ref, pos_ref, pacc_ref, logit_ref, wn0x_ref,
                wn0a_ref, bn0_ref, wn1_ref, bn1_ref, lmask_ref, oh3_ref,
                xout_ref, posout_ref):
    agg = aggp_ref[0] + aggp_ref[1]
    t = (jnp.dot(h_ref[...], wn0x_ref[...], preferred_element_type=_f32)
         + jnp.dot(agg, wn0a_ref[...], preferred_element_type=_f32)
         + bn0_ref[...])
    t = _silu(t)
    xout_ref[...] = (jnp.dot(t, wn1_ref[...], preferred_element_type=_f32)
                     + bn1_ref[...])
    acc = pacc_ref[0] + pacc_ref[1]
    deg = jnp.sum(acc * oh3_ref[...], axis=-1, keepdims=True)
    msg = acc * lmask_ref[...]
    gate = _sigmoid(logit_ref[...])
    upd = jnp.clip(gate * msg / jnp.maximum(deg, 1.0), -5.0, 5.0)
    posout_ref[...] = jnp.clip(pos_ref[...] + upd, -500.0, 500.0)


def _tc_node2(h, aggp, pos16, pacc, logit16, wn0x, wn0a, bn0, wn1, bn1,
              lmask, oh3):
    return pl.pallas_call(
        _node2_body,
        grid=(N // BN,),
        in_specs=[
            pl.BlockSpec((BN, H), lambda i: (i, 0)),
            pl.BlockSpec((NC, BN, H), lambda i: (0, i, 0)),
            pl.BlockSpec((BN, H), lambda i: (i, 0)),
            pl.BlockSpec((NC, BN, H), lambda i: (0, i, 0)),
            _w_spec((1, H)),
            _w_spec((H, H)),
            _w_spec((H, H)),
            _w_spec((1, H)),
            _w_spec((H, H)),
            _w_spec((1, H)),
            _w_spec((1, H)),
            _w_spec((1, H)),
        ],
        out_specs=[
            pl.BlockSpec((BN, H), lambda i: (i, 0)),
            pl.BlockSpec((BN, H), lambda i: (i, 0)),
        ],
        out_shape=[
            jax.ShapeDtypeStruct((N, H), _f32),
            jax.ShapeDtypeStruct((N, H), _f32),
        ],
    )(h, aggp, pos16, pacc, logit16, wn0x, wn0a, bn0, wn1, bn1, lmask, oh3)


def kernel(x, pos, edge_index, edge_attr, params, pos_scale_logit):
    src = edge_index[0]
    dst = edge_index[1]
    pos128 = jnp.zeros((N, H), _f32).at[:, :P].set(pos)
    z128 = jnp.zeros((N_PAD, H), _f32)
    lmask = jnp.zeros((1, 16), _f32).at[0, :P].set(1.0)
    lmask128 = jnp.zeros((1, H), _f32).at[0, :P].set(1.0)
    oh3 = jnp.zeros((1, H), _f32).at[0, P].set(1.0)
    logit128 = jnp.full((1, H), pos_scale_logit, _f32)

    lp1, lp2 = params

    def edge_w(lp):
        w0, b0 = lp['edge0']
        return (w0[:D], w0[D:2 * D], w0[2 * D:2 * D + 1], w0[2 * D + 1:],
                b0.reshape(1, H))

    wd1, ws1, wdist1, we1, b01 = edge_w(lp1)
    wd2, ws2, wdist2, we2, b02 = edge_w(lp2)
    w11, b11 = lp1['edge1'][0], lp1['edge1'][1].reshape(1, H)
    w12, b12 = lp2['edge1'][0], lp2['edge1'][1].reshape(1, H)
    wn0x1, wn0a1 = lp1['node0'][0][:D], lp1['node0'][0][D:]
    bn01 = lp1['node0'][1].reshape(1, H)
    wn11, bn11 = lp1['node1'][0], lp1['node1'][1].reshape(1, H)
    wn0x2, wn0a2 = lp2['node0'][0][:H], lp2['node0'][0][H:]
    bn02 = lp2['node0'][1].reshape(1, H)
    wn12, bn12 = lp2['node1'][0], lp2['node1'][1].reshape(1, H)
    wp0, bp0 = lp2['pos0'][0], lp2['pos0'][1].reshape(1, H)
    wp1 = lp2['pos1'][0].reshape(1, H)
    bp1 = jnp.broadcast_to(lp2['pos1'][1].reshape(1, 1), (1, H))

    npad = E_PAD - E
    dst_g = jnp.concatenate([dst, jnp.zeros((npad,), jnp.int32)])
    src_g = jnp.concatenate([src, jnp.zeros((npad,), jnp.int32)])
    dst2d = jnp.concatenate([dst, jnp.full((npad,), N, jnp.int32)])
    dst2d = dst2d.reshape(E_PAD // CS, CS)
    ea_p = jnp.concatenate([edge_attr, jnp.zeros((npad, ED), _f32)])

    posf = pos.T.reshape(-1)
    rel = _sc_rel(posf, dst_g, src_g)
    xd1, xs1 = _tc_proj(x, wd1, ws1)
    pre0_1 = _sc_gather(xd1, xs1, dst_g, src_g)
    m1 = _tc_edge1(pre0_1, rel, ea_p, wdist1, we1, b01, w11, b11, lmask)
    aggp1 = _sc_scatter(m1, dst2d, z128)
    h, xd2, xs2 = _tc_node1(x, aggp1, wn0x1, wn0a1, bn01, wn11, bn11,
                            wd2, ws2)

    pre0_2 = _sc_gather(xd2, xs2, dst_g, src_g)
    m2, relw = _tc_edge2(pre0_2, rel, ea_p, wdist2, we2, b02, w12, b12,
                         wp0, bp0, wp1, bp1, lmask, oh3)
    aggp2 = _sc_scatter(m2, dst2d, z128)
    pacc = _sc_scatter(relw, dst2d, z128)
    x_out, pos_out = _tc_node2(h, aggp2, pos128, pacc, logit128, wn0x2,
                               wn0a2, bn02, wn12, bn12, lmask128, oh3)
    return x_out, pos_out[:, :P]


# speedup vs baseline: 1.0640x; 1.0640x over previous
import functools

import jax
import jax.numpy as jnp
from jax import lax
from jax.experimental import pallas as pl
from jax.experimental.pallas import tpu as pltpu
from jax.experimental.pallas import tpu_sc as plsc

N, E, D, H, P, ED = 10000, 320000, 128, 128, 3, 16

NC = 2
NS = 16
NW = NC * NS
N_PAD = 10240
ROWS_PER_TILE = N_PAD // NS
E_PAD = 10240 * NW
EW = E_PAD // NW
CG = 128
NCG = EW // CG
CS = 128
NCS = EW // CS
IDX_ROWS = EW // CS
REL_CHUNK = 512
NREL = EW // REL_CHUNK

_f32 = jnp.float32

BE = 4096
BN = 2000


def _sigmoid(v):
    return 1.0 / (1.0 + jnp.exp(-v))


def _silu(v):
    return v * _sigmoid(v)


_SC_MESH = plsc.VectorSubcoreMesh(core_axis_name="c", subcore_axis_name="s",
                                  num_cores=NC, num_subcores=NS)


@functools.partial(
    pl.kernel,
    out_type=jax.ShapeDtypeStruct((E_PAD, 16), _f32),
    mesh=_SC_MESH,
    compiler_params=pltpu.CompilerParams(needs_layout_passes=False),
    scratch_types=[
        pltpu.VMEM((P * N,), _f32),
        pltpu.VMEM((EW,), jnp.int32),
        pltpu.VMEM((EW,), jnp.int32),
        pltpu.VMEM((REL_CHUNK, 16), _f32),
    ],
)
def _sc_rel(posf_hbm, dst_hbm, src_hbm, rel_hbm, posv, di, si, rbuf):
    wid = lax.axis_index("c") * NS + lax.axis_index("s")
    base0 = wid * EW
    pltpu.sync_copy(posf_hbm, posv)
    pltpu.sync_copy(dst_hbm.at[pl.ds(base0, EW)], di)
    pltpu.sync_copy(src_hbm.at[pl.ds(base0, EW)], si)
    zero16 = jnp.zeros((16,), _f32)

    @plsc.parallel_loop(0, REL_CHUNK)
    def _(r):
        rbuf[r, :] = zero16

    lanes = lax.iota(jnp.int32, 16)

    def chunk(j, carry):
        @plsc.parallel_loop(0, REL_CHUNK // 16, unroll=2)
        def _(v):
            e0 = j * REL_CHUNK + v * 16
            dstv = di[pl.ds(e0, 16)]
            srcv = si[pl.ds(e0, 16)]
            rows = v * 16 + lanes
            for comp in range(P):
                pdc = plsc.load_gather(posv, [dstv + comp * N])
                psc = plsc.load_gather(posv, [srcv + comp * N])
                cols = jnp.full((16,), comp, jnp.int32)
                plsc.store_scatter(rbuf, [rows, cols], pdc - psc)

        pltpu.sync_copy(rbuf, rel_hbm.at[pl.ds(base0 + j * REL_CHUNK, REL_CHUNK)])
        return carry

    lax.fori_loop(0, NREL, chunk, 0)


@functools.partial(
    pl.kernel,
    out_type=jax.ShapeDtypeStruct((E_PAD, H), _f32),
    mesh=_SC_MESH,
    compiler_params=pltpu.CompilerParams(needs_layout_passes=False),
    scratch_types=[
        pltpu.VMEM((NCG, CG), jnp.int32),
        pltpu.VMEM((NCG, CG), jnp.int32),
        pltpu.VMEM((2, CG, H), _f32),
        pltpu.VMEM((2, CG, H), _f32),
        pltpu.SemaphoreType.DMA,
        pltpu.SemaphoreType.DMA,
    ],
)
def _sc_gather(xd_hbm, xs_hbm, dst2d_hbm, src2d_hbm,
               pre0_hbm, di2, si2, bufd, bufs, sem0, sem1):
    wid = lax.axis_index("c") * NS + lax.axis_index("s")
    base0 = wid * EW
    pltpu.sync_copy(dst2d_hbm.at[pl.ds(wid * NCG, NCG)], di2)
    pltpu.sync_copy(src2d_hbm.at[pl.ds(wid * NCG, NCG)], si2)
    sems = (sem0, sem1)

    def gathers(j, slot, sem):
        cpd = pltpu.make_async_copy(
            xd_hbm.at[di2.at[j]], bufd.at[slot], sem)
        cps = pltpu.make_async_copy(
            xs_hbm.at[si2.at[j]], bufs.at[slot], sem)
        return cpd, cps

    for j0 in range(2):
        cpd, cps = gathers(j0, j0, sems[j0])
        cpd.start()
        cps.start()

    def pair(p, carry):
        for s2 in range(2):
            j = p * 2 + s2
            cpd, cps = gathers(j, s2, sems[s2])
            cpd.wait()
            cps.wait()

            @plsc.parallel_loop(0, CG, unroll=4)
            def _(r):
                for k in range(H // 16):
                    sl = pl.ds(k * 16, 16)
                    bufd[s2, r, sl] = bufd[s2, r, sl] + bufs[s2, r, sl]

            pltpu.sync_copy(bufd.at[s2],
                            pre0_hbm.at[pl.ds(base0 + j * CG, CG)])
            jj = j + 2

            @pl.when(jj < NCG)
            def _():
                cpd2, cps2 = gathers(jj, s2, sems[s2])
                cpd2.start()
                cps2.start()
        return carry

    lax.fori_loop(0, NCG // 2, pair, 0)


@functools.partial(
    pl.kernel,
    out_type=jax.ShapeDtypeStruct((NC, N_PAD, H), _f32),
    mesh=_SC_MESH,
    compiler_params=pltpu.CompilerParams(needs_layout_passes=False),
    scratch_types=[
        pltpu.VMEM_SHARED((N_PAD, H), _f32),
        pltpu.VMEM((IDX_ROWS, CS), jnp.int32),
        pltpu.VMEM((2, CS, H), _f32),
        pltpu.SemaphoreType.DMA,
        pltpu.SemaphoreType.DMA,
    ],
)
def _sc_scatter(m_hbm, dst2d_hbm, z_hbm, agg_hbm, shared, di2, mbuf,
                sem0, sem1):
    c = lax.axis_index("c")
    s = lax.axis_index("s")
    wid = c * NS + s
    rows = pl.ds(s * ROWS_PER_TILE, ROWS_PER_TILE)
    pltpu.sync_copy(z_hbm.at[rows], shared.at[rows])
    pltpu.sync_copy(dst2d_hbm.at[pl.ds(wid * IDX_ROWS, IDX_ROWS)], di2)
    plsc.subcore_barrier()
    sems = (sem0, sem1)
    base0 = wid * EW

    def load(j, slot, sem):
        return pltpu.make_async_copy(
            m_hbm.at[pl.ds(base0 + j * CS, CS)], mbuf.at[slot], sem)

    for j0 in range(2):
        load(j0, j0, sems[j0]).start()

    def pair(p, carry):
        for s2 in range(2):
            j = p * 2 + s2
            load(j, s2, sems[s2]).wait()
            pltpu.sync_copy(mbuf.at[s2], shared.at[di2.at[j]], add=True)
            jj = j + 2

            @pl.when(jj < NCS)
            def _():
                load(jj, s2, sems[s2]).start()
        return carry

    lax.fori_loop(0, NCS // 2, pair, 0)
    plsc.subcore_barrier()
    pltpu.sync_copy(shared.at[rows], agg_hbm.at[c, rows])


def _w_spec(shape):
    return pl.BlockSpec(shape, lambda i: (0,) * len(shape))


def _proj_body(x_ref, wd_ref, ws_ref, xd_ref, xs_ref):
    v = x_ref[...]
    xd_ref[...] = jnp.dot(v, wd_ref[...], preferred_element_type=_f32)
    xs_ref[...] = jnp.dot(v, ws_ref[...], preferred_element_type=_f32)


def _tc_proj(x, wd, ws):
    return pl.pallas_call(
        _proj_body,
        grid=(N // BN,),
        in_specs=[
            pl.BlockSpec((BN, D), lambda i: (i, 0)),
            _w_spec((D, H)),
            _w_spec((D, H)),
        ],
        out_specs=[
            pl.BlockSpec((BN, H), lambda i: (i, 0)),
            pl.BlockSpec((BN, H), lambda i: (i, 0)),
        ],
        out_shape=[
            jax.ShapeDtypeStruct((N, H), _f32),
            jax.ShapeDtypeStruct((N, H), _f32),
        ],
    )(x, wd, ws)


def _edge1_body(pre0_ref, rel_ref, ea_ref, wdist_ref, we_ref, b0_ref,
                w1_ref, b1_ref, lmask_ref, m_ref):
    rel = rel_ref[...]
    dist2 = jnp.sum(rel * rel * lmask_ref[...], axis=-1, keepdims=True)
    pre = (pre0_ref[...] + dist2 * wdist_ref[...] + b0_ref[...]
           + jnp.dot(ea_ref[...], we_ref[...], preferred_element_type=_f32))
    m1 = _silu(pre)
    z = jnp.dot(m1, w1_ref[...], preferred_element_type=_f32) + b1_ref[...]
    m_ref[...] = _silu(z)


def _tc_edge1(pre0, rel, ea, wdist, we, b0, w1, b1, lmask):
    return pl.pallas_call(
        _edge1_body,
        grid=(E_PAD // BE,),
        in_specs=[
            pl.BlockSpec((BE, H), lambda i: (i, 0)),
            pl.BlockSpec((BE, 16), lambda i: (i, 0)),
            pl.BlockSpec((BE, ED), lambda i: (i, 0)),
            _w_spec((1, H)),
            _w_spec((ED, H)),
            _w_spec((1, H)),
            _w_spec((H, H)),
            _w_spec((1, H)),
            _w_spec((1, 16)),
        ],
        out_specs=pl.BlockSpec((BE, H), lambda i: (i, 0)),
        out_shape=jax.ShapeDtypeStruct((E_PAD, H), _f32),
    )(pre0, rel, ea, wdist, we, b0, w1, b1, lmask)


def _edge2_body(pre0_ref, rel_ref, ea_ref, wdist_ref, we_ref, b0_ref,
                w1_ref, b1_ref, wp0_ref, bp0_ref, wp1_ref, bp1_ref,
                lmask_ref, oh3_ref, m_ref, relw_ref):
    rel = rel_ref[...]
    dist2 = jnp.sum(rel * rel * lmask_ref[...], axis=-1, keepdims=True)
    pre = (pre0_ref[...] + dist2 * wdist_ref[...] + b0_ref[...]
           + jnp.dot(ea_ref[...], we_ref[...], preferred_element_type=_f32))
    m1 = _silu(pre)
    z = jnp.dot(m1, w1_ref[...], preferred_element_type=_f32) + b1_ref[...]
    m = _silu(z)
    m_ref[...] = m
    t = jnp.dot(m, wp0_ref[...], preferred_element_type=_f32) + bp0_ref[...]
    t = _silu(t)
    w2 = jnp.sum(t * wp1_ref[...], axis=-1, keepdims=True) + bp1_ref[:, :1]
    relw = jnp.concatenate([rel * w2, jnp.zeros((BE, H - 16), _f32)], axis=1)
    relw_ref[...] = relw + oh3_ref[...]


def _tc_edge2(pre0, rel, ea, wdist, we, b0, w1, b1, wp0, bp0, wp1, bp1,
              lmask, oh3):
    return pl.pallas_call(
        _edge2_body,
        grid=(E_PAD // BE,),
        in_specs=[
            pl.BlockSpec((BE, H), lambda i: (i, 0)),
            pl.BlockSpec((BE, 16), lambda i: (i, 0)),
            pl.BlockSpec((BE, ED), lambda i: (i, 0)),
            _w_spec((1, H)),
            _w_spec((ED, H)),
            _w_spec((1, H)),
            _w_spec((H, H)),
            _w_spec((1, H)),
            _w_spec((H, H)),
            _w_spec((1, H)),
            _w_spec((1, H)),
            _w_spec((1, H)),
            _w_spec((1, 16)),
            _w_spec((1, H)),
        ],
        out_specs=[
            pl.BlockSpec((BE, H), lambda i: (i, 0)),
            pl.BlockSpec((BE, H), lambda i: (i, 0)),
        ],
        out_shape=[
            jax.ShapeDtypeStruct((E_PAD, H), _f32),
            jax.ShapeDtypeStruct((E_PAD, H), _f32),
        ],
    )(pre0, rel, ea, wdist, we, b0, w1, b1, wp0, bp0, wp1, bp1, lmask, oh3)


def _node1_body(x_ref, aggp_ref, wn0x_ref, wn0a_ref, bn0_ref, wn1_ref,
                bn1_ref, wd2_ref, ws2_ref, h_ref, xd2_ref, xs2_ref):
    agg = aggp_ref[0] + aggp_ref[1]
    t = (jnp.dot(x_ref[...], wn0x_ref[...], preferred_element_type=_f32)
         + jnp.dot(agg, wn0a_ref[...], preferred_element_type=_f32)
         + bn0_ref[...])
    t = _silu(t)
    hv = jnp.dot(t, wn1_ref[...], preferred_element_type=_f32) + bn1_ref[...]
    h_ref[...] = hv
    xd2_ref[...] = jnp.dot(hv, wd2_ref[...], preferred_element_type=_f32)
    xs2_ref[...] = jnp.dot(hv, ws2_ref[...], preferred_element_type=_f32)


def _tc_node1(x, aggp, wn0x, wn0a, bn0, wn1, bn1, wd2, ws2):
    return pl.pallas_call(
        _node1_body,
        grid=(N // BN,),
        in_specs=[
            pl.BlockSpec((BN, D), lambda i: (i, 0)),
            pl.BlockSpec((NC, BN, H), lambda i: (0, i, 0)),
            _w_spec((D, H)),
            _w_spec((H, H)),
            _w_spec((1, H)),
            _w_spec((H, H)),
            _w_spec((1, H)),
            _w_spec((H, H)),
            _w_spec((H, H)),
        ],
        out_specs=[
            pl.BlockSpec((BN, H), lambda i: (i, 0)),
            pl.BlockSpec((BN, H), lambda i: (i, 0)),
            pl.BlockSpec((BN, H), lambda i: (i, 0)),
        ],
        out_shape=[
            jax.ShapeDtypeStruct((N, H), _f32),
            jax.ShapeDtypeStruct((N, H), _f32),
            jax.ShapeDtypeStruct((N, H), _f32),
        ],
    )(x, aggp, wn0x, wn0a, bn0, wn1, bn1, wd2, ws2)


def _node2_body(h_ref, aggp_ref, pos_ref, pacc_ref, logit_ref, wn0x_ref,
                wn0a_ref, bn0_ref, wn1_ref, bn1_ref, lmask_ref, oh3_ref,
                xout_ref, posout_ref):
    agg = aggp_ref[0] + aggp_ref[1]
    t = (jnp.dot(h_ref[...], wn0x_ref[...], preferred_element_type=_f32)
         + jnp.dot(agg, wn0a_ref[...], preferred_element_type=_f32)
         + bn0_ref[...])
    t = _silu(t)
    xout_ref[...] = (jnp.dot(t, wn1_ref[...], preferred_element_type=_f32)
                     + bn1_ref[...])
    acc = pacc_ref[0] + pacc_ref[1]
    deg = jnp.sum(acc * oh3_ref[...], axis=-1, keepdims=True)
    msg = acc * lmask_ref[...]
    gate = _sigmoid(logit_ref[...])
    upd = jnp.clip(gate * msg / jnp.maximum(deg, 1.0), -5.0, 5.0)
    posout_ref[...] = jnp.clip(pos_ref[...] + upd, -500.0, 500.0)


def _tc_node2(h, aggp, pos16, pacc, logit16, wn0x, wn0a, bn0, wn1, bn1,
              lmask, oh3):
    return pl.pallas_call(
        _node2_body,
        grid=(N // BN,),
        in_specs=[
            pl.BlockSpec((BN, H), lambda i: (i, 0)),
            pl.BlockSpec((NC, BN, H), lambda i: (0, i, 0)),
            pl.BlockSpec((BN, H), lambda i: (i, 0)),
            pl.BlockSpec((NC, BN, H), lambda i: (0, i, 0)),
            _w_spec((1, H)),
            _w_spec((H, H)),
            _w_spec((H, H)),
            _w_spec((1, H)),
            _w_spec((H, H)),
            _w_spec((1, H)),
            _w_spec((1, H)),
            _w_spec((1, H)),
        ],
        out_specs=[
            pl.BlockSpec((BN, H), lambda i: (i, 0)),
            pl.BlockSpec((BN, H), lambda i: (i, 0)),
        ],
        out_shape=[
            jax.ShapeDtypeStruct((N, H), _f32),
            jax.ShapeDtypeStruct((N, H), _f32),
        ],
    )(h, aggp, pos16, pacc, logit16, wn0x, wn0a, bn0, wn1, bn1, lmask, oh3)


def kernel(x, pos, edge_index, edge_attr, params, pos_scale_logit):
    src = edge_index[0]
    dst = edge_index[1]
    pos128 = jnp.zeros((N, H), _f32).at[:, :P].set(pos)
    z128 = jnp.zeros((N_PAD, H), _f32)
    lmask = jnp.zeros((1, 16), _f32).at[0, :P].set(1.0)
    lmask128 = jnp.zeros((1, H), _f32).at[0, :P].set(1.0)
    oh3 = jnp.zeros((1, H), _f32).at[0, P].set(1.0)
    logit128 = jnp.full((1, H), pos_scale_logit, _f32)

    lp1, lp2 = params

    def edge_w(lp):
        w0, b0 = lp['edge0']
        return (w0[:D], w0[D:2 * D], w0[2 * D:2 * D + 1], w0[2 * D + 1:],
                b0.reshape(1, H))

    wd1, ws1, wdist1, we1, b01 = edge_w(lp1)
    wd2, ws2, wdist2, we2, b02 = edge_w(lp2)
    w11, b11 = lp1['edge1'][0], lp1['edge1'][1].reshape(1, H)
    w12, b12 = lp2['edge1'][0], lp2['edge1'][1].reshape(1, H)
    wn0x1, wn0a1 = lp1['node0'][0][:D], lp1['node0'][0][D:]
    bn01 = lp1['node0'][1].reshape(1, H)
    wn11, bn11 = lp1['node1'][0], lp1['node1'][1].reshape(1, H)
    wn0x2, wn0a2 = lp2['node0'][0][:H], lp2['node0'][0][H:]
    bn02 = lp2['node0'][1].reshape(1, H)
    wn12, bn12 = lp2['node1'][0], lp2['node1'][1].reshape(1, H)
    wp0, bp0 = lp2['pos0'][0], lp2['pos0'][1].reshape(1, H)
    wp1 = lp2['pos1'][0].reshape(1, H)
    bp1 = jnp.broadcast_to(lp2['pos1'][1].reshape(1, 1), (1, H))

    npad = E_PAD - E
    dst_g = jnp.concatenate([dst, jnp.zeros((npad,), jnp.int32)])
    src_g = jnp.concatenate([src, jnp.zeros((npad,), jnp.int32)])
    dst2d = jnp.concatenate([dst, jnp.full((npad,), N, jnp.int32)])
    dst2d = dst2d.reshape(E_PAD // CS, CS)
    dst2d_g = dst_g.reshape(E_PAD // CG, CG)
    src2d_g = src_g.reshape(E_PAD // CG, CG)
    ea_p = jnp.concatenate([edge_attr, jnp.zeros((npad, ED), _f32)])

    posf = pos.T.reshape(-1)
    rel = _sc_rel(posf, dst_g, src_g)
    xd1, xs1 = _tc_proj(x, wd1, ws1)
    pre0_1 = _sc_gather(xd1, xs1, dst2d_g, src2d_g)
    m1 = _tc_edge1(pre0_1, rel, ea_p, wdist1, we1, b01, w11, b11, lmask)
    aggp1 = _sc_scatter(m1, dst2d, z128)
    h, xd2, xs2 = _tc_node1(x, aggp1, wn0x1, wn0a1, bn01, wn11, bn11,
                            wd2, ws2)

    pre0_2 = _sc_gather(xd2, xs2, dst2d_g, src2d_g)
    m2, relw = _tc_edge2(pre0_2, rel, ea_p, wdist2, we2, b02, w12, b12,
                         wp0, bp0, wp1, bp1, lmask, oh3)
    aggp2 = _sc_scatter(m2, dst2d, z128)
    pacc = _sc_scatter(relw, dst2d, z128)
    x_out, pos_out = _tc_node2(h, aggp2, pos128, pacc, logit128, wn0x2,
                               wn0a2, bn02, wn12, bn12, lmask128, oh3)
    return x_out, pos_out[:, :P]


# speedup vs baseline: 1.6490x; 1.5499x over previous
import functools

import jax
import jax.numpy as jnp
from jax import lax
from jax.experimental import pallas as pl
from jax.experimental.pallas import tpu as pltpu
from jax.experimental.pallas import tpu_sc as plsc

N, E, D, H, P, ED = 10000, 320000, 128, 128, 3, 16

NC = 2
NS = 16
NW = NC * NS
N_PAD = 10240
ROWS_PER_TILE = N_PAD // NS
E_PAD = 10240 * NW
EW = E_PAD // NW
CG = 128
NCG = EW // CG
CS = 128
NCS = EW // CS
IDX_ROWS = EW // CS
REL_CHUNK = 512
NREL = EW // REL_CHUNK

_f32 = jnp.float32

BE = 4096
BN = 2000


def _sigmoid(v):
    return 1.0 / (1.0 + jnp.exp(-v))


def _silu(v):
    return v * _sigmoid(v)


_SC_MESH = plsc.VectorSubcoreMesh(core_axis_name="c", subcore_axis_name="s",
                                  num_cores=NC, num_subcores=NS)


@functools.partial(
    pl.kernel,
    out_type=jax.ShapeDtypeStruct((E_PAD, 16), _f32),
    mesh=_SC_MESH,
    compiler_params=pltpu.CompilerParams(needs_layout_passes=False),
    scratch_types=[
        pltpu.VMEM((P * N,), _f32),
        pltpu.VMEM((EW,), jnp.int32),
        pltpu.VMEM((EW,), jnp.int32),
        pltpu.VMEM((REL_CHUNK, 16), _f32),
    ],
)
def _sc_rel(posf_hbm, dst_hbm, src_hbm, rel_hbm, posv, di, si, rbuf):
    wid = lax.axis_index("c") * NS + lax.axis_index("s")
    base0 = wid * EW
    pltpu.sync_copy(posf_hbm, posv)
    pltpu.sync_copy(dst_hbm.at[pl.ds(base0, EW)], di)
    pltpu.sync_copy(src_hbm.at[pl.ds(base0, EW)], si)
    zero16 = jnp.zeros((16,), _f32)

    @plsc.parallel_loop(0, REL_CHUNK)
    def _(r):
        rbuf[r, :] = zero16

    lanes = lax.iota(jnp.int32, 16)

    def chunk(j, carry):
        @plsc.parallel_loop(0, REL_CHUNK // 16, unroll=2)
        def _(v):
            e0 = j * REL_CHUNK + v * 16
            dstv = di[pl.ds(e0, 16)]
            srcv = si[pl.ds(e0, 16)]
            rows = v * 16 + lanes
            for comp in range(P):
                pdc = plsc.load_gather(posv, [dstv + comp * N])
                psc = plsc.load_gather(posv, [srcv + comp * N])
                cols = jnp.full((16,), comp, jnp.int32)
                plsc.store_scatter(rbuf, [rows, cols], pdc - psc)

        pltpu.sync_copy(rbuf, rel_hbm.at[pl.ds(base0 + j * REL_CHUNK, REL_CHUNK)])
        return carry

    lax.fori_loop(0, NREL, chunk, 0)


@functools.partial(
    pl.kernel,
    out_type=jax.ShapeDtypeStruct((E_PAD, H), _f32),
    mesh=_SC_MESH,
    compiler_params=pltpu.CompilerParams(needs_layout_passes=False),
    scratch_types=[
        pltpu.VMEM((NCG, CG), jnp.int32),
        pltpu.VMEM((NCG, CG), jnp.int32),
        pltpu.VMEM((2, CG, H), _f32),
        pltpu.VMEM((2, CG, H), _f32),
        pltpu.SemaphoreType.DMA,
        pltpu.SemaphoreType.DMA,
    ],
)
def _sc_gather(xd_hbm, xs_hbm, dst2d_hbm, src2d_hbm,
               pre0_hbm, di2, si2, bufd, bufs, sem0, sem1):
    wid = lax.axis_index("c") * NS + lax.axis_index("s")
    base0 = wid * EW
    pltpu.sync_copy(dst2d_hbm.at[pl.ds(wid * NCG, NCG)], di2)
    pltpu.sync_copy(src2d_hbm.at[pl.ds(wid * NCG, NCG)], si2)
    sems = (sem0, sem1)

    def gathers(j, slot, sem):
        cpd = pltpu.make_async_copy(
            xd_hbm.at[di2.at[j]], bufd.at[slot], sem)
        cps = pltpu.make_async_copy(
            xs_hbm.at[si2.at[j]], bufs.at[slot], sem)
        return cpd, cps

    for j0 in range(2):
        cpd, cps = gathers(j0, j0, sems[j0])
        cpd.start()
        cps.start()

    def pair(p, carry):
        for s2 in range(2):
            j = p * 2 + s2
            cpd, cps = gathers(j, s2, sems[s2])
            cpd.wait()
            cps.wait()

            @plsc.parallel_loop(0, CG, unroll=4)
            def _(r):
                for k in range(H // 16):
                    sl = pl.ds(k * 16, 16)
                    bufd[s2, r, sl] = bufd[s2, r, sl] + bufs[s2, r, sl]

            pltpu.sync_copy(bufd.at[s2],
                            pre0_hbm.at[pl.ds(base0 + j * CG, CG)])
            jj = j + 2

            @pl.when(jj < NCG)
            def _():
                cpd2, cps2 = gathers(jj, s2, sems[s2])
                cpd2.start()
                cps2.start()
        return carry

    lax.fori_loop(0, NCG // 2, pair, 0)


@functools.partial(
    pl.kernel,
    out_type=jax.ShapeDtypeStruct((NC, N_PAD, H), _f32),
    mesh=_SC_MESH,
    compiler_params=pltpu.CompilerParams(needs_layout_passes=False),
    scratch_types=[
        pltpu.VMEM_SHARED((N_PAD, H), _f32),
        pltpu.VMEM((IDX_ROWS, CS), jnp.int32),
        pltpu.VMEM((2, CS, H), _f32),
        pltpu.SemaphoreType.DMA,
        pltpu.SemaphoreType.DMA,
    ],
)
def _sc_scatter(m_hbm, dst2d_hbm, z_hbm, agg_hbm, shared, di2, mbuf,
                sem0, sem1):
    c = lax.axis_index("c")
    s = lax.axis_index("s")
    wid = c * NS + s
    rows = pl.ds(s * ROWS_PER_TILE, ROWS_PER_TILE)
    pltpu.sync_copy(z_hbm.at[rows], shared.at[rows])
    pltpu.sync_copy(dst2d_hbm.at[pl.ds(wid * IDX_ROWS, IDX_ROWS)], di2)
    plsc.subcore_barrier()
    sems = (sem0, sem1)
    base0 = wid * EW

    def load(j, slot, sem):
        return pltpu.make_async_copy(
            m_hbm.at[pl.ds(base0 + j * CS, CS)], mbuf.at[slot], sem)

    for j0 in range(2):
        load(j0, j0, sems[j0]).start()

    def pair(p, carry):
        for s2 in range(2):
            j = p * 2 + s2
            load(j, s2, sems[s2]).wait()
            pltpu.sync_copy(mbuf.at[s2], shared.at[di2.at[j]], add=True)
            jj = j + 2

            @pl.when(jj < NCS)
            def _():
                load(jj, s2, sems[s2]).start()
        return carry

    lax.fori_loop(0, NCS // 2, pair, 0)
    plsc.subcore_barrier()
    pltpu.sync_copy(shared.at[rows], agg_hbm.at[c, rows])


def _w_spec(shape):
    return pl.BlockSpec(shape, lambda i: (0,) * len(shape))


def _proj_body(x_ref, wd_ref, ws_ref, xd_ref, xs_ref):
    v = x_ref[...]
    xd_ref[...] = jnp.dot(v, wd_ref[...], preferred_element_type=_f32)
    xs_ref[...] = jnp.dot(v, ws_ref[...], preferred_element_type=_f32)


def _tc_proj(x, wd, ws):
    return pl.pallas_call(
        _proj_body,
        grid=(N // BN,),
        in_specs=[
            pl.BlockSpec((BN, D), lambda i: (i, 0)),
            _w_spec((D, H)),
            _w_spec((D, H)),
        ],
        out_specs=[
            pl.BlockSpec((BN, H), lambda i: (i, 0)),
            pl.BlockSpec((BN, H), lambda i: (i, 0)),
        ],
        out_shape=[
            jax.ShapeDtypeStruct((N, H), _f32),
            jax.ShapeDtypeStruct((N, H), _f32),
        ],
    )(x, wd, ws)


def _edge1_body(pre0_ref, rel_ref, ea_ref, wdist_ref, we_ref, b0_ref,
                w1_ref, b1_ref, lmask_ref, m_ref):
    rel = rel_ref[...]
    dist2 = jnp.sum(rel * rel * lmask_ref[...], axis=-1, keepdims=True)
    pre = (pre0_ref[...] + dist2 * wdist_ref[...] + b0_ref[...]
           + jnp.dot(ea_ref[...], we_ref[...], preferred_element_type=_f32))
    m1 = _silu(pre)
    z = jnp.dot(m1, w1_ref[...], preferred_element_type=_f32) + b1_ref[...]
    m_ref[...] = _silu(z)


def _tc_edge1(pre0, rel, ea, wdist, we, b0, w1, b1, lmask):
    return pl.pallas_call(
        _edge1_body,
        grid=(E_PAD // BE,),
        in_specs=[
            pl.BlockSpec((BE, H), lambda i: (i, 0)),
            pl.BlockSpec((BE, 16), lambda i: (i, 0)),
            pl.BlockSpec((BE, ED), lambda i: (i, 0)),
            _w_spec((1, H)),
            _w_spec((ED, H)),
            _w_spec((1, H)),
            _w_spec((H, H)),
            _w_spec((1, H)),
            _w_spec((1, 16)),
        ],
        out_specs=pl.BlockSpec((BE, H), lambda i: (i, 0)),
        out_shape=jax.ShapeDtypeStruct((E_PAD, H), _f32),
    )(pre0, rel, ea, wdist, we, b0, w1, b1, lmask)


def _edge2_body(pre0_ref, rel_ref, ea_ref, wdist_ref, we_ref, b0_ref,
                w1_ref, b1_ref, wp0_ref, bp0_ref, wp1_ref, bp1_ref,
                lmask_ref, oh3_ref, m_ref, relw_ref):
    rel = rel_ref[...]
    dist2 = jnp.sum(rel * rel * lmask_ref[...], axis=-1, keepdims=True)
    pre = (pre0_ref[...] + dist2 * wdist_ref[...] + b0_ref[...]
           + jnp.dot(ea_ref[...], we_ref[...], preferred_element_type=_f32))
    m1 = _silu(pre)
    z = jnp.dot(m1, w1_ref[...], preferred_element_type=_f32) + b1_ref[...]
    m = _silu(z)
    m_ref[...] = m
    t = jnp.dot(m, wp0_ref[...], preferred_element_type=_f32) + bp0_ref[...]
    t = _silu(t)
    w2 = jnp.sum(t * wp1_ref[...], axis=-1, keepdims=True) + bp1_ref[:, :1]
    relw = jnp.concatenate([rel * w2, jnp.zeros((BE, H - 16), _f32)], axis=1)
    relw_ref[...] = relw + oh3_ref[...]


def _tc_edge2(pre0, rel, ea, wdist, we, b0, w1, b1, wp0, bp0, wp1, bp1,
              lmask, oh3):
    return pl.pallas_call(
        _edge2_body,
        grid=(E_PAD // BE,),
        in_specs=[
            pl.BlockSpec((BE, H), lambda i: (i, 0)),
            pl.BlockSpec((BE, 16), lambda i: (i, 0)),
            pl.BlockSpec((BE, ED), lambda i: (i, 0)),
            _w_spec((1, H)),
            _w_spec((ED, H)),
            _w_spec((1, H)),
            _w_spec((H, H)),
            _w_spec((1, H)),
            _w_spec((H, H)),
            _w_spec((1, H)),
            _w_spec((1, H)),
            _w_spec((1, H)),
            _w_spec((1, 16)),
            _w_spec((1, H)),
        ],
        out_specs=[
            pl.BlockSpec((BE, H), lambda i: (i, 0)),
            pl.BlockSpec((BE, H), lambda i: (i, 0)),
        ],
        out_shape=[
            jax.ShapeDtypeStruct((E_PAD, H), _f32),
            jax.ShapeDtypeStruct((E_PAD, H), _f32),
        ],
    )(pre0, rel, ea, wdist, we, b0, w1, b1, wp0, bp0, wp1, bp1, lmask, oh3)


def _node1_body(x_ref, aggp_ref, wn0x_ref, wn0a_ref, bn0_ref, wn1_ref,
                bn1_ref, wd2_ref, ws2_ref, h_ref, xd2_ref, xs2_ref):
    agg = aggp_ref[0] + aggp_ref[1]
    t = (jnp.dot(x_ref[...], wn0x_ref[...], preferred_element_type=_f32)
         + jnp.dot(agg, wn0a_ref[...], preferred_element_type=_f32)
         + bn0_ref[...])
    t = _silu(t)
    hv = jnp.dot(t, wn1_ref[...], preferred_element_type=_f32) + bn1_ref[...]
    h_ref[...] = hv
    xd2_ref[...] = jnp.dot(hv, wd2_ref[...], preferred_element_type=_f32)
    xs2_ref[...] = jnp.dot(hv, ws2_ref[...], preferred_element_type=_f32)


def _tc_node1(x, aggp, wn0x, wn0a, bn0, wn1, bn1, wd2, ws2):
    return pl.pallas_call(
        _node1_body,
        grid=(N // BN,),
        in_specs=[
            pl.BlockSpec((BN, D), lambda i: (i, 0)),
            pl.BlockSpec((NC, BN, H), lambda i: (0, i, 0)),
            _w_spec((D, H)),
            _w_spec((H, H)),
            _w_spec((1, H)),
            _w_spec((H, H)),
            _w_spec((1, H)),
            _w_spec((H, H)),
            _w_spec((H, H)),
        ],
        out_specs=[
            pl.BlockSpec((BN, H), lambda i: (i, 0)),
            pl.BlockSpec((BN, H), lambda i: (i, 0)),
            pl.BlockSpec((BN, H), lambda i: (i, 0)),
        ],
        out_shape=[
            jax.ShapeDtypeStruct((N, H), _f32),
            jax.ShapeDtypeStruct((N, H), _f32),
            jax.ShapeDtypeStruct((N, H), _f32),
        ],
    )(x, aggp, wn0x, wn0a, bn0, wn1, bn1, wd2, ws2)


def _node2_body(h_ref, aggp_ref, pos_ref, pacc_ref, logit_ref, wn0x_ref,
                wn0a_ref, bn0_ref, wn1_ref, bn1_ref, lmask_ref, oh3_ref,
                xout_ref, posout_ref):
    agg = aggp_ref[0] + aggp_ref[1]
    t = (jnp.dot(h_ref[...], wn0x_ref[...], preferred_element_type=_f32)
         + jnp.dot(agg, wn0a_ref[...], preferred_element_type=_f32)
         + bn0_ref[...])
    t = _silu(t)
    xout_ref[...] = (jnp.dot(t, wn1_ref[...], preferred_element_type=_f32)
                     + bn1_ref[...])
    acc = pacc_ref[0] + pacc_ref[1]
    deg = jnp.sum(acc * oh3_ref[...], axis=-1, keepdims=True)
    msg = acc * lmask_ref[...]
    gate = _sigmoid(logit_ref[...])
    upd = jnp.clip(gate * msg / jnp.maximum(deg, 1.0), -5.0, 5.0)
    posout_ref[...] = jnp.clip(pos_ref[...] + upd, -500.0, 500.0)


def _tc_node2(h, aggp, pos16, pacc, logit16, wn0x, wn0a, bn0, wn1, bn1,
              lmask, oh3):
    return pl.pallas_call(
        _node2_body,
        grid=(N // BN,),
        in_specs=[
            pl.BlockSpec((BN, H), lambda i: (i, 0)),
            pl.BlockSpec((NC, BN, H), lambda i: (0, i, 0)),
            pl.BlockSpec((BN, H), lambda i: (i, 0)),
            pl.BlockSpec((NC, BN, H), lambda i: (0, i, 0)),
            _w_spec((1, H)),
            _w_spec((H, H)),
            _w_spec((H, H)),
            _w_spec((1, H)),
            _w_spec((H, H)),
            _w_spec((1, H)),
            _w_spec((1, H)),
            _w_spec((1, H)),
        ],
        out_specs=[
            pl.BlockSpec((BN, H), lambda i: (i, 0)),
            pl.BlockSpec((BN, H), lambda i: (i, 0)),
        ],
        out_shape=[
            jax.ShapeDtypeStruct((N, H), _f32),
            jax.ShapeDtypeStruct((N, H), _f32),
        ],
    )(h, aggp, pos16, pacc, logit16, wn0x, wn0a, bn0, wn1, bn1, lmask, oh3)


def kernel(x, pos, edge_index, edge_attr, params, pos_scale_logit):
    src = edge_index[0]
    dst = edge_index[1]
    pos128 = jnp.zeros((N, H), _f32).at[:, :P].set(pos)
    z128 = jnp.zeros((N_PAD, H), _f32)
    lmask = jnp.zeros((1, 16), _f32).at[0, :P].set(1.0)
    lmask128 = jnp.zeros((1, H), _f32).at[0, :P].set(1.0)
    oh3 = jnp.zeros((1, H), _f32).at[0, P].set(1.0)
    logit128 = jnp.full((1, H), pos_scale_logit, _f32)

    lp1, lp2 = params

    def edge_w(lp):
        w0, b0 = lp['edge0']
        return (w0[:D], w0[D:2 * D], w0[2 * D:2 * D + 1], w0[2 * D + 1:],
                b0.reshape(1, H))

    wd1, ws1, wdist1, we1, b01 = edge_w(lp1)
    wd2, ws2, wdist2, we2, b02 = edge_w(lp2)
    w11, b11 = lp1['edge1'][0], lp1['edge1'][1].reshape(1, H)
    w12, b12 = lp2['edge1'][0], lp2['edge1'][1].reshape(1, H)
    wn0x1, wn0a1 = lp1['node0'][0][:D], lp1['node0'][0][D:]
    bn01 = lp1['node0'][1].reshape(1, H)
    wn11, bn11 = lp1['node1'][0], lp1['node1'][1].reshape(1, H)
    wn0x2, wn0a2 = lp2['node0'][0][:H], lp2['node0'][0][H:]
    bn02 = lp2['node0'][1].reshape(1, H)
    wn12, bn12 = lp2['node1'][0], lp2['node1'][1].reshape(1, H)
    wp0, bp0 = lp2['pos0'][0], lp2['pos0'][1].reshape(1, H)
    wp1 = lp2['pos1'][0].reshape(1, H)
    bp1 = jnp.broadcast_to(lp2['pos1'][1].reshape(1, 1), (1, H))

    npad = E_PAD - E
    padg = jnp.arange(npad, dtype=jnp.int32) % N
    pads = N + jnp.arange(npad, dtype=jnp.int32) % (N_PAD - N)
    dst_g = jnp.concatenate([dst, padg])
    src_g = jnp.concatenate([src, padg])
    dst2d = jnp.concatenate([dst, pads])
    dst2d = dst2d.reshape(E_PAD // CS, CS)
    dst2d_g = dst_g.reshape(E_PAD // CG, CG)
    src2d_g = src_g.reshape(E_PAD // CG, CG)
    ea_p = jnp.concatenate([edge_attr, jnp.zeros((npad, ED), _f32)])

    posf = pos.T.reshape(-1)
    rel = _sc_rel(posf, dst_g, src_g)
    xd1, xs1 = _tc_proj(x, wd1, ws1)
    pre0_1 = _sc_gather(xd1, xs1, dst2d_g, src2d_g)
    m1 = _tc_edge1(pre0_1, rel, ea_p, wdist1, we1, b01, w11, b11, lmask)
    aggp1 = _sc_scatter(m1, dst2d, z128)
    h, xd2, xs2 = _tc_node1(x, aggp1, wn0x1, wn0a1, bn01, wn11, bn11,
                            wd2, ws2)

    pre0_2 = _sc_gather(xd2, xs2, dst2d_g, src2d_g)
    m2, relw = _tc_edge2(pre0_2, rel, ea_p, wdist2, we2, b02, w12, b12,
                         wp0, bp0, wp1, bp1, lmask, oh3)
    aggp2 = _sc_scatter(m2, dst2d, z128)
    pacc = _sc_scatter(relw, dst2d, z128)
    x_out, pos_out = _tc_node2(h, aggp2, pos128, pacc, logit128, wn0x2,
                               wn0a2, bn02, wn12, bn12, lmask128, oh3)
    return x_out, pos_out[:, :P]


# speedup vs baseline: 1.6501x; 1.0007x over previous
import functools

import jax
import jax.numpy as jnp
from jax import lax
from jax.experimental import pallas as pl
from jax.experimental.pallas import tpu as pltpu
from jax.experimental.pallas import tpu_sc as plsc

N, E, D, H, P, ED = 10000, 320000, 128, 128, 3, 16

NC = 2
NS = 16
NW = NC * NS
N_PAD = 10240
ROWS_PER_TILE = N_PAD // NS
E_PAD = 10240 * NW
EW = E_PAD // NW
CG = 128
NCG = EW // CG
CS = 128
NCS = EW // CS
IDX_ROWS = EW // CS
REL_CHUNK = 512
NREL = EW // REL_CHUNK

_f32 = jnp.float32

BE = 4096
BN = 2000


def _sigmoid(v):
    return 1.0 / (1.0 + jnp.exp(-v))


def _silu(v):
    return v * _sigmoid(v)


_SC_MESH = plsc.VectorSubcoreMesh(core_axis_name="c", subcore_axis_name="s",
                                  num_cores=NC, num_subcores=NS)


@functools.partial(
    pl.kernel,
    out_type=jax.ShapeDtypeStruct((E_PAD, 16), _f32),
    mesh=_SC_MESH,
    compiler_params=pltpu.CompilerParams(needs_layout_passes=False),
    scratch_types=[
        pltpu.VMEM((P * N,), _f32),
        pltpu.VMEM((EW,), jnp.int32),
        pltpu.VMEM((EW,), jnp.int32),
        pltpu.VMEM((REL_CHUNK, 16), _f32),
    ],
)
def _sc_rel(posf_hbm, dst_hbm, src_hbm, rel_hbm, posv, di, si, rbuf):
    wid = lax.axis_index("c") * NS + lax.axis_index("s")
    base0 = wid * EW
    pltpu.sync_copy(posf_hbm, posv)
    pltpu.sync_copy(dst_hbm.at[pl.ds(base0, EW)], di)
    pltpu.sync_copy(src_hbm.at[pl.ds(base0, EW)], si)
    zero16 = jnp.zeros((16,), _f32)

    @plsc.parallel_loop(0, REL_CHUNK)
    def _(r):
        rbuf[r, :] = zero16

    lanes = lax.iota(jnp.int32, 16)

    def chunk(j, carry):
        @plsc.parallel_loop(0, REL_CHUNK // 16, unroll=2)
        def _(v):
            e0 = j * REL_CHUNK + v * 16
            dstv = di[pl.ds(e0, 16)]
            srcv = si[pl.ds(e0, 16)]
            rows = v * 16 + lanes
            for comp in range(P):
                pdc = plsc.load_gather(posv, [dstv + comp * N])
                psc = plsc.load_gather(posv, [srcv + comp * N])
                cols = jnp.full((16,), comp, jnp.int32)
                plsc.store_scatter(rbuf, [rows, cols], pdc - psc)

        pltpu.sync_copy(rbuf, rel_hbm.at[pl.ds(base0 + j * REL_CHUNK, REL_CHUNK)])
        return carry

    lax.fori_loop(0, NREL, chunk, 0)


@functools.partial(
    pl.kernel,
    out_type=jax.ShapeDtypeStruct((E_PAD, H), _f32),
    mesh=_SC_MESH,
    compiler_params=pltpu.CompilerParams(needs_layout_passes=False),
    scratch_types=[
        pltpu.VMEM((NCG, CG), jnp.int32),
        pltpu.VMEM((NCG, CG), jnp.int32),
        pltpu.VMEM((2, CG, H), _f32),
        pltpu.VMEM((2, CG, H), _f32),
        pltpu.SemaphoreType.DMA,
        pltpu.SemaphoreType.DMA,
    ],
)
def _sc_gather(xd_hbm, xs_hbm, dst2d_hbm, src2d_hbm,
               pre0_hbm, di2, si2, bufd, bufs, sem0, sem1):
    wid = lax.axis_index("c") * NS + lax.axis_index("s")
    base0 = wid * EW
    pltpu.sync_copy(dst2d_hbm.at[pl.ds(wid * NCG, NCG)], di2)
    pltpu.sync_copy(src2d_hbm.at[pl.ds(wid * NCG, NCG)], si2)
    sems = (sem0, sem1)

    def gathers(j, slot, sem):
        cpd = pltpu.make_async_copy(
            xd_hbm.at[di2.at[j]], bufd.at[slot], sem)
        cps = pltpu.make_async_copy(
            xs_hbm.at[si2.at[j]], bufs.at[slot], sem)
        return cpd, cps

    for j0 in range(2):
        cpd, cps = gathers(j0, j0, sems[j0])
        cpd.start()
        cps.start()

    def pair(p, carry):
        for s2 in range(2):
            j = p * 2 + s2
            cpd, cps = gathers(j, s2, sems[s2])
            cpd.wait()
            cps.wait()

            @plsc.parallel_loop(0, CG, unroll=4)
            def _(r):
                for k in range(H // 16):
                    sl = pl.ds(k * 16, 16)
                    bufd[s2, r, sl] = bufd[s2, r, sl] + bufs[s2, r, sl]

            pltpu.sync_copy(bufd.at[s2],
                            pre0_hbm.at[pl.ds(base0 + j * CG, CG)])
            jj = j + 2

            @pl.when(jj < NCG)
            def _():
                cpd2, cps2 = gathers(jj, s2, sems[s2])
                cpd2.start()
                cps2.start()
        return carry

    lax.fori_loop(0, NCG // 2, pair, 0)


@functools.partial(
    pl.kernel,
    out_type=jax.ShapeDtypeStruct((NC, N_PAD, H), _f32),
    mesh=_SC_MESH,
    compiler_params=pltpu.CompilerParams(needs_layout_passes=False),
    scratch_types=[
        pltpu.VMEM_SHARED((N_PAD, H), _f32),
        pltpu.VMEM((IDX_ROWS, CS), jnp.int32),
        pltpu.VMEM((2, CS, H), _f32),
        pltpu.SemaphoreType.DMA,
        pltpu.SemaphoreType.DMA,
    ],
)
def _sc_scatter(m_hbm, dst2d_hbm, z_hbm, agg_hbm, shared, di2, mbuf,
                sem0, sem1):
    c = lax.axis_index("c")
    s = lax.axis_index("s")
    wid = c * NS + s
    rows = pl.ds(s * ROWS_PER_TILE, ROWS_PER_TILE)
    pltpu.sync_copy(z_hbm.at[rows], shared.at[rows])
    pltpu.sync_copy(dst2d_hbm.at[pl.ds(wid * IDX_ROWS, IDX_ROWS)], di2)
    plsc.subcore_barrier()
    sems = (sem0, sem1)
    base0 = wid * EW

    def load(j, slot, sem):
        return pltpu.make_async_copy(
            m_hbm.at[pl.ds(base0 + j * CS, CS)], mbuf.at[slot], sem)

    for j0 in range(2):
        load(j0, j0, sems[j0]).start()

    def pair(p, carry):
        for s2 in range(2):
            j = p * 2 + s2
            load(j, s2, sems[s2]).wait()
            pltpu.sync_copy(mbuf.at[s2], shared.at[di2.at[j]], add=True)
            jj = j + 2

            @pl.when(jj < NCS)
            def _():
                load(jj, s2, sems[s2]).start()
        return carry

    lax.fori_loop(0, NCS // 2, pair, 0)
    plsc.subcore_barrier()
    pltpu.sync_copy(shared.at[rows], agg_hbm.at[c, rows])


def _w_spec(shape):
    return pl.BlockSpec(shape, lambda i: (0,) * len(shape))


def _proj_body(x_ref, wd_ref, ws_ref, xd_ref, xs_ref):
    v = x_ref[...]
    xd_ref[...] = jnp.dot(v, wd_ref[...], preferred_element_type=_f32)
    xs_ref[...] = jnp.dot(v, ws_ref[...], preferred_element_type=_f32)


def _tc_proj(x, wd, ws):
    return pl.pallas_call(
        _proj_body,
        grid=(N // BN,),
        in_specs=[
            pl.BlockSpec((BN, D), lambda i: (i, 0)),
            _w_spec((D, H)),
            _w_spec((D, H)),
        ],
        out_specs=[
            pl.BlockSpec((BN, H), lambda i: (i, 0)),
            pl.BlockSpec((BN, H), lambda i: (i, 0)),
        ],
        out_shape=[
            jax.ShapeDtypeStruct((N, H), _f32),
            jax.ShapeDtypeStruct((N, H), _f32),
        ],
    )(x, wd, ws)


def _edge1_body(pre0_ref, rel_ref, ea_ref, wdist_ref, we_ref, b0_ref,
                w1_ref, b1_ref, lmask_ref, m_ref):
    rel = rel_ref[...]
    dist2 = jnp.sum(rel * rel * lmask_ref[...], axis=-1, keepdims=True)
    pre = (pre0_ref[...] + dist2 * wdist_ref[...] + b0_ref[...]
           + jnp.dot(ea_ref[...], we_ref[...], preferred_element_type=_f32))
    m1 = _silu(pre)
    z = jnp.dot(m1, w1_ref[...], preferred_element_type=_f32) + b1_ref[...]
    m_ref[...] = _silu(z)


def _tc_edge1(pre0, rel, ea, wdist, we, b0, w1, b1, lmask):
    return pl.pallas_call(
        _edge1_body,
        grid=(E_PAD // BE,),
        in_specs=[
            pl.BlockSpec((BE, H), lambda i: (i, 0)),
            pl.BlockSpec((BE, 16), lambda i: (i, 0)),
            pl.BlockSpec((BE, ED), lambda i: (i, 0)),
            _w_spec((1, H)),
            _w_spec((ED, H)),
            _w_spec((1, H)),
            _w_spec((H, H)),
            _w_spec((1, H)),
            _w_spec((1, 16)),
        ],
        out_specs=pl.BlockSpec((BE, H), lambda i: (i, 0)),
        out_shape=jax.ShapeDtypeStruct((E_PAD, H), _f32),
    )(pre0, rel, ea, wdist, we, b0, w1, b1, lmask)


def _edge2_body(pre0_ref, rel_ref, ea_ref, wdist_ref, we_ref, b0_ref,
                w1_ref, b1_ref, wp0_ref, bp0_ref, wp1_ref, bp1_ref,
                lmask_ref, oh3_ref, m_ref, relw_ref):
    rel = rel_ref[...]
    dist2 = jnp.sum(rel * rel * lmask_ref[...], axis=-1, keepdims=True)
    pre = (pre0_ref[...] + dist2 * wdist_ref[...] + b0_ref[...]
           + jnp.dot(ea_ref[...], we_ref[...], preferred_element_type=_f32))
    m1 = _silu(pre)
    z = jnp.dot(m1, w1_ref[...], preferred_element_type=_f32) + b1_ref[...]
    m = _silu(z)
    m_ref[...] = m
    t = jnp.dot(m, wp0_ref[...], preferred_element_type=_f32) + bp0_ref[...]
    t = _silu(t)
    w2 = jnp.sum(t * wp1_ref[...], axis=-1, keepdims=True) + bp1_ref[:, :1]
    relw = jnp.concatenate([rel * w2, jnp.zeros((BE, H - 16), _f32)], axis=1)
    relw_ref[...] = relw + oh3_ref[...]


def _tc_edge2(pre0, rel, ea, wdist, we, b0, w1, b1, wp0, bp0, wp1, bp1,
              lmask, oh3):
    return pl.pallas_call(
        _edge2_body,
        grid=(E_PAD // BE,),
        in_specs=[
            pl.BlockSpec((BE, H), lambda i: (i, 0)),
            pl.BlockSpec((BE, 16), lambda i: (i, 0)),
            pl.BlockSpec((BE, ED), lambda i: (i, 0)),
            _w_spec((1, H)),
            _w_spec((ED, H)),
            _w_spec((1, H)),
            _w_spec((H, H)),
            _w_spec((1, H)),
            _w_spec((H, H)),
            _w_spec((1, H)),
            _w_spec((1, H)),
            _w_spec((1, H)),
            _w_spec((1, 16)),
            _w_spec((1, H)),
        ],
        out_specs=[
            pl.BlockSpec((BE, H), lambda i: (i, 0)),
            pl.BlockSpec((BE, H), lambda i: (i, 0)),
        ],
        out_shape=[
            jax.ShapeDtypeStruct((E_PAD, H), _f32),
            jax.ShapeDtypeStruct((E_PAD, H), _f32),
        ],
    )(pre0, rel, ea, wdist, we, b0, w1, b1, wp0, bp0, wp1, bp1, lmask, oh3)


def _node1_body(x_ref, aggp_ref, wn0x_ref, wn0a_ref, bn0_ref, wn1_ref,
                bn1_ref, wd2_ref, ws2_ref, h_ref, xd2_ref, xs2_ref):
    agg = aggp_ref[0] + aggp_ref[1]
    t = (jnp.dot(x_ref[...], wn0x_ref[...], preferred_element_type=_f32)
         + jnp.dot(agg, wn0a_ref[...], preferred_element_type=_f32)
         + bn0_ref[...])
    t = _silu(t)
    hv = jnp.dot(t, wn1_ref[...], preferred_element_type=_f32) + bn1_ref[...]
    h_ref[...] = hv
    xd2_ref[...] = jnp.dot(hv, wd2_ref[...], preferred_element_type=_f32)
    xs2_ref[...] = jnp.dot(hv, ws2_ref[...], preferred_element_type=_f32)


def _tc_node1(x, aggp, wn0x, wn0a, bn0, wn1, bn1, wd2, ws2):
    return pl.pallas_call(
        _node1_body,
        grid=(N // BN,),
        in_specs=[
            pl.BlockSpec((BN, D), lambda i: (i, 0)),
            pl.BlockSpec((NC, BN, H), lambda i: (0, i, 0)),
            _w_spec((D, H)),
            _w_spec((H, H)),
            _w_spec((1, H)),
            _w_spec((H, H)),
            _w_spec((1, H)),
            _w_spec((H, H)),
            _w_spec((H, H)),
        ],
        out_specs=[
            pl.BlockSpec((BN, H), lambda i: (i, 0)),
            pl.BlockSpec((BN, H), lambda i: (i, 0)),
            pl.BlockSpec((BN, H), lambda i: (i, 0)),
        ],
        out_shape=[
            jax.ShapeDtypeStruct((N, H), _f32),
            jax.ShapeDtypeStruct((N, H), _f32),
            jax.ShapeDtypeStruct((N, H), _f32),
        ],
    )(x, aggp, wn0x, wn0a, bn0, wn1, bn1, wd2, ws2)


def _node2_body(h_ref, aggp_ref, pos_ref, pacc_ref, logit_ref, wn0x_ref,
                wn0a_ref, bn0_ref, wn1_ref, bn1_ref, lmask_ref, oh3_ref,
                xout_ref, posout_ref):
    agg = aggp_ref[0] + aggp_ref[1]
    t = (jnp.dot(h_ref[...], wn0x_ref[...], preferred_element_type=_f32)
         + jnp.dot(agg, wn0a_ref[...], preferred_element_type=_f32)
         + bn0_ref[...])
    t = _silu(t)
    xout_ref[...] = (jnp.dot(t, wn1_ref[...], preferred_element_type=_f32)
                     + bn1_ref[...])
    acc = pacc_ref[0] + pacc_ref[1]
    deg = jnp.sum(acc * oh3_ref[...], axis=-1, keepdims=True)
    msg = acc * lmask_ref[...]
    gate = _sigmoid(logit_ref[...])
    upd = jnp.clip(gate * msg / jnp.maximum(deg, 1.0), -5.0, 5.0)
    posout_ref[...] = jnp.clip(pos_ref[...] + upd, -500.0, 500.0)


def _tc_node2(h, aggp, pos16, pacc, logit16, wn0x, wn0a, bn0, wn1, bn1,
              lmask, oh3):
    return pl.pallas_call(
        _node2_body,
        grid=(N // BN,),
        in_specs=[
            pl.BlockSpec((BN, H), lambda i: (i, 0)),
            pl.BlockSpec((NC, BN, H), lambda i: (0, i, 0)),
            pl.BlockSpec((BN, H), lambda i: (i, 0)),
            pl.BlockSpec((NC, BN, H), lambda i: (0, i, 0)),
            _w_spec((1, H)),
            _w_spec((H, H)),
            _w_spec((H, H)),
            _w_spec((1, H)),
            _w_spec((H, H)),
            _w_spec((1, H)),
            _w_spec((1, H)),
            _w_spec((1, H)),
        ],
        out_specs=[
            pl.BlockSpec((BN, H), lambda i: (i, 0)),
            pl.BlockSpec((BN, H), lambda i: (i, 0)),
        ],
        out_shape=[
            jax.ShapeDtypeStruct((N, H), _f32),
            jax.ShapeDtypeStruct((N, H), _f32),
        ],
    )(h, aggp, pos16, pacc, logit16, wn0x, wn0a, bn0, wn1, bn1, lmask, oh3)


def kernel(x, pos, edge_index, edge_attr, params, pos_scale_logit):
    src = edge_index[0]
    dst = edge_index[1]
    pos128 = jnp.zeros((N, H), _f32).at[:, :P].set(pos)
    z128 = jnp.zeros((N_PAD, H), _f32)
    lmask = jnp.zeros((1, 16), _f32).at[0, :P].set(1.0)
    lmask128 = jnp.zeros((1, H), _f32).at[0, :P].set(1.0)
    oh3_16 = jnp.zeros((1, 16), _f32).at[0, P].set(1.0)
    oh3 = jnp.zeros((1, H), _f32).at[0, P].set(1.0)
    logit128 = jnp.full((1, H), pos_scale_logit, _f32)

    lp1, lp2 = params

    def edge_w(lp):
        w0, b0 = lp['edge0']
        return (w0[:D], w0[D:2 * D], w0[2 * D:2 * D + 1], w0[2 * D + 1:],
                b0.reshape(1, H))

    wd1, ws1, wdist1, we1, b01 = edge_w(lp1)
    wd2, ws2, wdist2, we2, b02 = edge_w(lp2)
    w11, b11 = lp1['edge1'][0], lp1['edge1'][1].reshape(1, H)
    w12, b12 = lp2['edge1'][0], lp2['edge1'][1].reshape(1, H)
    wn0x1, wn0a1 = lp1['node0'][0][:D], lp1['node0'][0][D:]
    bn01 = lp1['node0'][1].reshape(1, H)
    wn11, bn11 = lp1['node1'][0], lp1['node1'][1].reshape(1, H)
    wn0x2, wn0a2 = lp2['node0'][0][:H], lp2['node0'][0][H:]
    bn02 = lp2['node0'][1].reshape(1, H)
    wn12, bn12 = lp2['node1'][0], lp2['node1'][1].reshape(1, H)
    wp0, bp0 = lp2['pos0'][0], lp2['pos0'][1].reshape(1, H)
    wp1 = lp2['pos1'][0].reshape(1, H)
    bp1 = jnp.broadcast_to(lp2['pos1'][1].reshape(1, 1), (1, H))

    npad = E_PAD - E
    padg = jnp.arange(npad, dtype=jnp.int32) % N
    pads = N + jnp.arange(npad, dtype=jnp.int32) % (N_PAD - N)
    dst_g = jnp.concatenate([dst, padg])
    src_g = jnp.concatenate([src, padg])
    dst2d = jnp.concatenate([dst, pads])
    dst2d = dst2d.reshape(E_PAD // CS, CS)
    dst2d_g = dst_g.reshape(E_PAD // CG, CG)
    src2d_g = src_g.reshape(E_PAD // CG, CG)
    ea_p = jnp.concatenate([edge_attr, jnp.zeros((npad, ED), _f32)])

    posf = pos.T.reshape(-1)
    rel = _sc_rel(posf, dst_g, src_g)
    xd1, xs1 = _tc_proj(x, wd1, ws1)
    pre0_1 = _sc_gather(xd1, xs1, dst2d_g, src2d_g)
    m1 = _tc_edge1(pre0_1, rel, ea_p, wdist1, we1, b01, w11, b11, lmask)
    aggp1 = _sc_scatter(m1, dst2d, z128)
    h, xd2, xs2 = _tc_node1(x, aggp1, wn0x1, wn0a1, bn01, wn11, bn11,
                            wd2, ws2)

    pre0_2 = _sc_gather(xd2, xs2, dst2d_g, src2d_g)
    m2, relw = _tc_edge2(pre0_2, rel, ea_p, wdist2, we2, b02, w12, b12,
                         wp0, bp0, wp1, bp1, lmask, oh3)
    aggp2 = _sc_scatter(m2, dst2d, z128)
    pacc = _sc_scatter(relw, dst2d, z128)
    x_out, pos_out = _tc_node2(h, aggp2, pos128, pacc, logit128, wn0x2,
                               wn0a2, bn02, wn12, bn12, lmask128, oh3)
    return x_out, pos_out[:, :P]


# speedup vs baseline: 1.6904x; 1.0244x over previous
import functools

import jax
import jax.numpy as jnp
from jax import lax
from jax.experimental import pallas as pl
from jax.experimental.pallas import tpu as pltpu
from jax.experimental.pallas import tpu_sc as plsc

N, E, D, H, P, ED = 10000, 320000, 128, 128, 3, 16

NC = 2
NS = 16
NW = NC * NS
N_PAD = 10240
ROWS_PER_TILE = N_PAD // NS
E_PAD = 10240 * NW
EW = E_PAD // NW
CG = 128
CS = 128
REL_CHUNK = 512
NREL = EW // REL_CHUNK
E_HALF = E_PAD // 2
EWH = E_HALF // NW
NCGH = EWH // CG
NCSH = EWH // CS

_f32 = jnp.float32

BE = 4096
NBE_H = E_HALF // BE
BN = 2000


def _sigmoid(v):
    return 1.0 / (1.0 + jnp.exp(-v))


def _silu(v):
    return v * _sigmoid(v)


_SC_MESH = plsc.VectorSubcoreMesh(core_axis_name="c", subcore_axis_name="s",
                                  num_cores=NC, num_subcores=NS)


@functools.partial(
    pl.kernel,
    out_type=jax.ShapeDtypeStruct((E_PAD, 16), _f32),
    mesh=_SC_MESH,
    compiler_params=pltpu.CompilerParams(needs_layout_passes=False),
    scratch_types=[
        pltpu.VMEM((P * N,), _f32),
        pltpu.VMEM((EW,), jnp.int32),
        pltpu.VMEM((EW,), jnp.int32),
        pltpu.VMEM((REL_CHUNK, 16), _f32),
    ],
)
def _sc_rel(posf_hbm, dst_hbm, src_hbm, rel_hbm, posv, di, si, rbuf):
    wid = lax.axis_index("c") * NS + lax.axis_index("s")
    base0 = wid * EW
    pltpu.sync_copy(posf_hbm, posv)
    pltpu.sync_copy(dst_hbm.at[pl.ds(base0, EW)], di)
    pltpu.sync_copy(src_hbm.at[pl.ds(base0, EW)], si)
    zero16 = jnp.zeros((16,), _f32)

    @plsc.parallel_loop(0, REL_CHUNK)
    def _(r):
        rbuf[r, :] = zero16

    lanes = lax.iota(jnp.int32, 16)

    def chunk(j, carry):
        @plsc.parallel_loop(0, REL_CHUNK // 16, unroll=2)
        def _(v):
            e0 = j * REL_CHUNK + v * 16
            dstv = di[pl.ds(e0, 16)]
            srcv = si[pl.ds(e0, 16)]
            rows = v * 16 + lanes
            for comp in range(P):
                pdc = plsc.load_gather(posv, [dstv + comp * N])
                psc = plsc.load_gather(posv, [srcv + comp * N])
                cols = jnp.full((16,), comp, jnp.int32)
                plsc.store_scatter(rbuf, [rows, cols], pdc - psc)

        pltpu.sync_copy(rbuf, rel_hbm.at[pl.ds(base0 + j * REL_CHUNK, REL_CHUNK)])
        return carry

    lax.fori_loop(0, NREL, chunk, 0)


def _make_gather(half):
    @functools.partial(
        pl.kernel,
        out_type=jax.ShapeDtypeStruct((E_HALF, H), _f32),
        mesh=_SC_MESH,
        compiler_params=pltpu.CompilerParams(needs_layout_passes=False),
        scratch_types=[
            pltpu.VMEM((NCGH, CG), jnp.int32),
            pltpu.VMEM((NCGH, CG), jnp.int32),
            pltpu.VMEM((2, CG, H), _f32),
            pltpu.VMEM((2, CG, H), _f32),
            pltpu.SemaphoreType.DMA,
            pltpu.SemaphoreType.DMA,
        ],
        name=f"sc_gather_h{half}",
    )
    def gather_k(xd_hbm, xs_hbm, dst2d_hbm, src2d_hbm,
                 pre0_hbm, di2, si2, bufd, bufs, sem0, sem1):
        wid = lax.axis_index("c") * NS + lax.axis_index("s")
        base0 = wid * EWH
        idxrow0 = half * (E_HALF // CG) + wid * NCGH
        pltpu.sync_copy(dst2d_hbm.at[pl.ds(idxrow0, NCGH)], di2)
        pltpu.sync_copy(src2d_hbm.at[pl.ds(idxrow0, NCGH)], si2)
        sems = (sem0, sem1)

        def gathers(j, slot, sem):
            cpd = pltpu.make_async_copy(
                xd_hbm.at[di2.at[j]], bufd.at[slot], sem)
            cps = pltpu.make_async_copy(
                xs_hbm.at[si2.at[j]], bufs.at[slot], sem)
            return cpd, cps

        for j0 in range(2):
            cpd, cps = gathers(j0, j0, sems[j0])
            cpd.start()
            cps.start()

        def pair(p, carry):
            for s2 in range(2):
                j = p * 2 + s2
                cpd, cps = gathers(j, s2, sems[s2])
                cpd.wait()
                cps.wait()

                @plsc.parallel_loop(0, CG, unroll=4)
                def _(r):
                    for k in range(H // 16):
                        sl = pl.ds(k * 16, 16)
                        bufd[s2, r, sl] = bufd[s2, r, sl] + bufs[s2, r, sl]

                pltpu.sync_copy(bufd.at[s2],
                                pre0_hbm.at[pl.ds(base0 + j * CG, CG)])
                jj = j + 2

                @pl.when(jj < NCGH)
                def _():
                    cpd2, cps2 = gathers(jj, s2, sems[s2])
                    cpd2.start()
                    cps2.start()
            return carry

        lax.fori_loop(0, NCGH // 2, pair, 0)

    return gather_k


_sc_gather_h = (_make_gather(0), _make_gather(1))


def _make_scatter(half):
    @functools.partial(
        pl.kernel,
        out_type=jax.ShapeDtypeStruct((NC, N_PAD, H), _f32),
        mesh=_SC_MESH,
        compiler_params=pltpu.CompilerParams(needs_layout_passes=False),
        scratch_types=[
            pltpu.VMEM_SHARED((N_PAD, H), _f32),
            pltpu.VMEM((NCSH, CS), jnp.int32),
            pltpu.VMEM((2, CS, H), _f32),
            pltpu.SemaphoreType.DMA,
            pltpu.SemaphoreType.DMA,
        ],
        name=f"sc_scatter_h{half}",
    )
    def scatter_k(m_hbm, dst2d_hbm, z_hbm, agg_hbm, shared, di2, mbuf,
                  sem0, sem1):
        c = lax.axis_index("c")
        s = lax.axis_index("s")
        wid = c * NS + s
        rows = pl.ds(s * ROWS_PER_TILE, ROWS_PER_TILE)
        idxrow0 = half * (E_HALF // CS) + wid * NCSH
        pltpu.sync_copy(z_hbm.at[rows], shared.at[rows])
        pltpu.sync_copy(dst2d_hbm.at[pl.ds(idxrow0, NCSH)], di2)
        plsc.subcore_barrier()
        sems = (sem0, sem1)
        base0 = wid * EWH

        def load(j, slot, sem):
            return pltpu.make_async_copy(
                m_hbm.at[pl.ds(base0 + j * CS, CS)], mbuf.at[slot], sem)

        for j0 in range(2):
            load(j0, j0, sems[j0]).start()

        def pair(p, carry):
            for s2 in range(2):
                j = p * 2 + s2
                load(j, s2, sems[s2]).wait()
                pltpu.sync_copy(mbuf.at[s2], shared.at[di2.at[j]], add=True)
                jj = j + 2

                @pl.when(jj < NCSH)
                def _():
                    load(jj, s2, sems[s2]).start()
            return carry

        lax.fori_loop(0, NCSH // 2, pair, 0)
        plsc.subcore_barrier()
        pltpu.sync_copy(shared.at[rows], agg_hbm.at[c, rows])

    return scatter_k


_sc_scatter_h = (_make_scatter(0), _make_scatter(1))


def _w_spec(shape):
    return pl.BlockSpec(shape, lambda i: (0,) * len(shape))


def _proj_body(x_ref, wd_ref, ws_ref, xd_ref, xs_ref):
    v = x_ref[...]
    xd_ref[...] = jnp.dot(v, wd_ref[...], preferred_element_type=_f32)
    xs_ref[...] = jnp.dot(v, ws_ref[...], preferred_element_type=_f32)


def _tc_proj(x, wd, ws):
    return pl.pallas_call(
        _proj_body,
        grid=(N // BN,),
        in_specs=[
            pl.BlockSpec((BN, D), lambda i: (i, 0)),
            _w_spec((D, H)),
            _w_spec((D, H)),
        ],
        out_specs=[
            pl.BlockSpec((BN, H), lambda i: (i, 0)),
            pl.BlockSpec((BN, H), lambda i: (i, 0)),
        ],
        out_shape=[
            jax.ShapeDtypeStruct((N, H), _f32),
            jax.ShapeDtypeStruct((N, H), _f32),
        ],
    )(x, wd, ws)


def _edge1_body(pre0_ref, rel_ref, ea_ref, wdist_ref, we_ref, b0_ref,
                w1_ref, b1_ref, lmask_ref, m_ref):
    rel = rel_ref[...]
    dist2 = jnp.sum(rel * rel * lmask_ref[...], axis=-1, keepdims=True)
    pre = (pre0_ref[...] + dist2 * wdist_ref[...] + b0_ref[...]
           + jnp.dot(ea_ref[...], we_ref[...], preferred_element_type=_f32))
    m1 = _silu(pre)
    z = jnp.dot(m1, w1_ref[...], preferred_element_type=_f32) + b1_ref[...]
    m_ref[...] = _silu(z)


def _tc_edge1(half, pre0, rel, ea, wdist, we, b0, w1, b1, lmask):
    off = half * NBE_H
    return pl.pallas_call(
        _edge1_body,
        grid=(NBE_H,),
        in_specs=[
            pl.BlockSpec((BE, H), lambda i: (i, 0)),
            pl.BlockSpec((BE, 16), lambda i: (i + off, 0)),
            pl.BlockSpec((BE, ED), lambda i: (i + off, 0)),
            _w_spec((1, H)),
            _w_spec((ED, H)),
            _w_spec((1, H)),
            _w_spec((H, H)),
            _w_spec((1, H)),
            _w_spec((1, 16)),
        ],
        out_specs=pl.BlockSpec((BE, H), lambda i: (i, 0)),
        out_shape=jax.ShapeDtypeStruct((E_HALF, H), _f32),
    )(pre0, rel, ea, wdist, we, b0, w1, b1, lmask)


def _edge2_body(pre0_ref, rel_ref, ea_ref, wdist_ref, we_ref, b0_ref,
                w1_ref, b1_ref, wp0_ref, bp0_ref, wp1_ref, bp1_ref,
                lmask_ref, oh3_ref, m_ref, relw_ref):
    rel = rel_ref[...]
    dist2 = jnp.sum(rel * rel * lmask_ref[...], axis=-1, keepdims=True)
    pre = (pre0_ref[...] + dist2 * wdist_ref[...] + b0_ref[...]
           + jnp.dot(ea_ref[...], we_ref[...], preferred_element_type=_f32))
    m1 = _silu(pre)
    z = jnp.dot(m1, w1_ref[...], preferred_element_type=_f32) + b1_ref[...]
    m = _silu(z)
    m_ref[...] = m
    t = jnp.dot(m, wp0_ref[...], preferred_element_type=_f32) + bp0_ref[...]
    t = _silu(t)
    w2 = jnp.sum(t * wp1_ref[...], axis=-1, keepdims=True) + bp1_ref[:, :1]
    relw = jnp.concatenate([rel * w2, jnp.zeros((BE, H - 16), _f32)], axis=1)
    relw_ref[...] = relw + oh3_ref[...]


def _tc_edge2(half, pre0, rel, ea, wdist, we, b0, w1, b1, wp0, bp0, wp1, bp1,
              lmask, oh3):
    off = half * NBE_H
    return pl.pallas_call(
        _edge2_body,
        grid=(NBE_H,),
        in_specs=[
            pl.BlockSpec((BE, H), lambda i: (i, 0)),
            pl.BlockSpec((BE, 16), lambda i: (i + off, 0)),
            pl.BlockSpec((BE, ED), lambda i: (i + off, 0)),
            _w_spec((1, H)),
            _w_spec((ED, H)),
            _w_spec((1, H)),
            _w_spec((H, H)),
            _w_spec((1, H)),
            _w_spec((H, H)),
            _w_spec((1, H)),
            _w_spec((1, H)),
            _w_spec((1, H)),
            _w_spec((1, 16)),
            _w_spec((1, H)),
        ],
        out_specs=[
            pl.BlockSpec((BE, H), lambda i: (i, 0)),
            pl.BlockSpec((BE, H), lambda i: (i, 0)),
        ],
        out_shape=[
            jax.ShapeDtypeStruct((E_HALF, H), _f32),
            jax.ShapeDtypeStruct((E_HALF, H), _f32),
        ],
    )(pre0, rel, ea, wdist, we, b0, w1, b1, wp0, bp0, wp1, bp1, lmask, oh3)


def _node1_body(x_ref, aggpa_ref, aggpb_ref, wn0x_ref, wn0a_ref, bn0_ref,
                wn1_ref, bn1_ref, wd2_ref, ws2_ref, h_ref, xd2_ref, xs2_ref):
    agg = (aggpa_ref[0] + aggpa_ref[1]) + (aggpb_ref[0] + aggpb_ref[1])
    t = (jnp.dot(x_ref[...], wn0x_ref[...], preferred_element_type=_f32)
         + jnp.dot(agg, wn0a_ref[...], preferred_element_type=_f32)
         + bn0_ref[...])
    t = _silu(t)
    hv = jnp.dot(t, wn1_ref[...], preferred_element_type=_f32) + bn1_ref[...]
    h_ref[...] = hv
    xd2_ref[...] = jnp.dot(hv, wd2_ref[...], preferred_element_type=_f32)
    xs2_ref[...] = jnp.dot(hv, ws2_ref[...], preferred_element_type=_f32)


def _tc_node1(x, aggpa, aggpb, wn0x, wn0a, bn0, wn1, bn1, wd2, ws2):
    return pl.pallas_call(
        _node1_body,
        grid=(N // BN,),
        in_specs=[
            pl.BlockSpec((BN, D), lambda i: (i, 0)),
            pl.BlockSpec((NC, BN, H), lambda i: (0, i, 0)),
            pl.BlockSpec((NC, BN, H), lambda i: (0, i, 0)),
            _w_spec((D, H)),
            _w_spec((H, H)),
            _w_spec((1, H)),
            _w_spec((H, H)),
            _w_spec((1, H)),
            _w_spec((H, H)),
            _w_spec((H, H)),
        ],
        out_specs=[
            pl.BlockSpec((BN, H), lambda i: (i, 0)),
            pl.BlockSpec((BN, H), lambda i: (i, 0)),
            pl.BlockSpec((BN, H), lambda i: (i, 0)),
        ],
        out_shape=[
            jax.ShapeDtypeStruct((N, H), _f32),
            jax.ShapeDtypeStruct((N, H), _f32),
            jax.ShapeDtypeStruct((N, H), _f32),
        ],
    )(x, aggpa, aggpb, wn0x, wn0a, bn0, wn1, bn1, wd2, ws2)


def _node2_body(h_ref, aggpa_ref, aggpb_ref, pos_ref, pacca_ref, paccb_ref,
                logit_ref, wn0x_ref, wn0a_ref, bn0_ref, wn1_ref, bn1_ref,
                lmask_ref, oh3_ref, xout_ref, posout_ref):
    agg = (aggpa_ref[0] + aggpa_ref[1]) + (aggpb_ref[0] + aggpb_ref[1])
    t = (jnp.dot(h_ref[...], wn0x_ref[...], preferred_element_type=_f32)
         + jnp.dot(agg, wn0a_ref[...], preferred_element_type=_f32)
         + bn0_ref[...])
    t = _silu(t)
    xout_ref[...] = (jnp.dot(t, wn1_ref[...], preferred_element_type=_f32)
                     + bn1_ref[...])
    acc = (pacca_ref[0] + pacca_ref[1]) + (paccb_ref[0] + paccb_ref[1])
    deg = jnp.sum(acc * oh3_ref[...], axis=-1, keepdims=True)
    msg = acc * lmask_ref[...]
    gate = _sigmoid(logit_ref[...])
    upd = jnp.clip(gate * msg / jnp.maximum(deg, 1.0), -5.0, 5.0)
    posout_ref[...] = jnp.clip(pos_ref[...] + upd, -500.0, 500.0)


def _tc_node2(h, aggpa, aggpb, pos128, pacca, paccb, logit128, wn0x, wn0a,
              bn0, wn1, bn1, lmask, oh3):
    return pl.pallas_call(
        _node2_body,
        grid=(N // BN,),
        in_specs=[
            pl.BlockSpec((BN, H), lambda i: (i, 0)),
            pl.BlockSpec((NC, BN, H), lambda i: (0, i, 0)),
            pl.BlockSpec((NC, BN, H), lambda i: (0, i, 0)),
            pl.BlockSpec((BN, H), lambda i: (i, 0)),
            pl.BlockSpec((NC, BN, H), lambda i: (0, i, 0)),
            pl.BlockSpec((NC, BN, H), lambda i: (0, i, 0)),
            _w_spec((1, H)),
            _w_spec((H, H)),
            _w_spec((H, H)),
            _w_spec((1, H)),
            _w_spec((H, H)),
            _w_spec((1, H)),
            _w_spec((1, H)),
            _w_spec((1, H)),
        ],
        out_specs=[
            pl.BlockSpec((BN, H), lambda i: (i, 0)),
            pl.BlockSpec((BN, H), lambda i: (i, 0)),
        ],
        out_shape=[
            jax.ShapeDtypeStruct((N, H), _f32),
            jax.ShapeDtypeStruct((N, H), _f32),
        ],
    )(h, aggpa, aggpb, pos128, pacca, paccb, logit128, wn0x, wn0a, bn0,
      wn1, bn1, lmask, oh3)


def kernel(x, pos, edge_index, edge_attr, params, pos_scale_logit):
    src = edge_index[0]
    dst = edge_index[1]
    pos128 = jnp.zeros((N, H), _f32).at[:, :P].set(pos)
    z128 = jnp.zeros((N_PAD, H), _f32)
    lmask = jnp.zeros((1, 16), _f32).at[0, :P].set(1.0)
    lmask128 = jnp.zeros((1, H), _f32).at[0, :P].set(1.0)
    oh3 = jnp.zeros((1, H), _f32).at[0, P].set(1.0)
    logit128 = jnp.full((1, H), pos_scale_logit, _f32)

    lp1, lp2 = params

    def edge_w(lp):
        w0, b0 = lp['edge0']
        return (w0[:D], w0[D:2 * D], w0[2 * D:2 * D + 1], w0[2 * D + 1:],
                b0.reshape(1, H))

    wd1, ws1, wdist1, we1, b01 = edge_w(lp1)
    wd2, ws2, wdist2, we2, b02 = edge_w(lp2)
    w11, b11 = lp1['edge1'][0], lp1['edge1'][1].reshape(1, H)
    w12, b12 = lp2['edge1'][0], lp2['edge1'][1].reshape(1, H)
    wn0x1, wn0a1 = lp1['node0'][0][:D], lp1['node0'][0][D:]
    bn01 = lp1['node0'][1].reshape(1, H)
    wn11, bn11 = lp1['node1'][0], lp1['node1'][1].reshape(1, H)
    wn0x2, wn0a2 = lp2['node0'][0][:H], lp2['node0'][0][H:]
    bn02 = lp2['node0'][1].reshape(1, H)
    wn12, bn12 = lp2['node1'][0], lp2['node1'][1].reshape(1, H)
    wp0, bp0 = lp2['pos0'][0], lp2['pos0'][1].reshape(1, H)
    wp1 = lp2['pos1'][0].reshape(1, H)
    bp1 = jnp.broadcast_to(lp2['pos1'][1].reshape(1, 1), (1, H))

    npad = E_PAD - E
    padg = jnp.arange(npad, dtype=jnp.int32) % N
    pads = N + jnp.arange(npad, dtype=jnp.int32) % (N_PAD - N)
    dst_g = jnp.concatenate([dst, padg])
    src_g = jnp.concatenate([src, padg])
    dst2d = jnp.concatenate([dst, pads])
    dst2d = dst2d.reshape(E_PAD // CS, CS)
    dst2d_g = dst_g.reshape(E_PAD // CG, CG)
    src2d_g = src_g.reshape(E_PAD // CG, CG)
    ea_p = jnp.concatenate([edge_attr, jnp.zeros((npad, ED), _f32)])

    posf = pos.T.reshape(-1)
    rel = _sc_rel(posf, dst_g, src_g)
    xd1, xs1 = _tc_proj(x, wd1, ws1)
    pre0a = _sc_gather_h[0](xd1, xs1, dst2d_g, src2d_g)
    pre0b = _sc_gather_h[1](xd1, xs1, dst2d_g, src2d_g)
    ma = _tc_edge1(0, pre0a, rel, ea_p, wdist1, we1, b01, w11, b11, lmask)
    mb = _tc_edge1(1, pre0b, rel, ea_p, wdist1, we1, b01, w11, b11, lmask)
    aggpa = _sc_scatter_h[0](ma, dst2d, z128)
    aggpb = _sc_scatter_h[1](mb, dst2d, z128)
    h, xd2, xs2 = _tc_node1(x, aggpa, aggpb, wn0x1, wn0a1, bn01, wn11, bn11,
                            wd2, ws2)

    pre2a = _sc_gather_h[0](xd2, xs2, dst2d_g, src2d_g)
    pre2b = _sc_gather_h[1](xd2, xs2, dst2d_g, src2d_g)
    m2a, relwa = _tc_edge2(0, pre2a, rel, ea_p, wdist2, we2, b02, w12, b12,
                           wp0, bp0, wp1, bp1, lmask, oh3)
    m2b, relwb = _tc_edge2(1, pre2b, rel, ea_p, wdist2, we2, b02, w12, b12,
                           wp0, bp0, wp1, bp1, lmask, oh3)
    aggp2a = _sc_scatter_h[0](m2a, dst2d, z128)
    aggp2b = _sc_scatter_h[1](m2b, dst2d, z128)
    pacca = _sc_scatter_h[0](relwa, dst2d, z128)
    paccb = _sc_scatter_h[1](relwb, dst2d, z128)
    x_out, pos_out = _tc_node2(h, aggp2a, aggp2b, pos128, pacca, paccb,
                               logit128, wn0x2, wn0a2, bn02, wn12, bn12,
                               lmask128, oh3)
    return x_out, pos_out[:, :P]
